# Initial kernel scaffold; baseline (speedup 1.0000x reference)
import functools

import jax
import jax.numpy as jnp
from jax import lax
from jax.experimental import pallas as pl
from jax.experimental.pallas import tpu as pltpu
from jax.experimental.pallas import tpu_sc as plsc

L = 16
NCORES = 2
NSUB = 16
NW = NCORES * NSUB


def _mesh():
    return plsc.VectorSubcoreMesh(core_axis_name="c", subcore_axis_name="s")


def _tc_matmul(a, w, pre_bias=None, pre_relu=False, bm=1000):
    m, k = a.shape
    _, n = w.shape
    assert m % bm == 0

    def body(*refs):
        if pre_bias is None:
            a_ref, w_ref, o_ref = refs
            av = a_ref[...]
        else:
            a_ref, w_ref, b_ref, o_ref = refs
            av = a_ref[...] + b_ref[...]
        if pre_relu:
            av = jnp.maximum(av, 0.0)
        o_ref[...] = jnp.dot(av, w_ref[...], preferred_element_type=jnp.float32)

    in_specs = [
        pl.BlockSpec((bm, k), lambda i: (i, 0)),
        pl.BlockSpec((k, n), lambda i: (0, 0)),
    ]
    args = [a, w]
    if pre_bias is not None:
        in_specs.append(pl.BlockSpec((1, k), lambda i: (0, 0)))
        args.append(pre_bias.reshape(1, k))
    return pl.pallas_call(
        body,
        grid=(m // bm,),
        in_specs=in_specs,
        out_specs=pl.BlockSpec((bm, n), lambda i: (i, 0)),
        out_shape=jax.ShapeDtypeStruct((m, n), jnp.float32),
    )(*args)


def _tc_decode_mlp(h0, h1, bc, w1a, w1b, b1, w2, b2, w3, b3, w4p, b4p, bm=4096):
    m, k = h0.shape

    def body(h0_ref, h1_ref, bc_ref, w1a_ref, w1b_ref, b1_ref, w2_ref, b2_ref,
             w3_ref, b3_ref, w4_ref, b4_ref, o_ref):
        a0 = h0_ref[...] + bc_ref[...]
        a1 = h1_ref[...] + bc_ref[...]
        h = jnp.dot(a0, w1a_ref[...], preferred_element_type=jnp.float32)
        h += jnp.dot(a1, w1b_ref[...], preferred_element_type=jnp.float32)
        h = jnp.maximum(h + b1_ref[...], 0.0)
        h = jnp.maximum(jnp.dot(h, w2_ref[...], preferred_element_type=jnp.float32) + b2_ref[...], 0.0)
        h = jnp.maximum(jnp.dot(h, w3_ref[...], preferred_element_type=jnp.float32) + b3_ref[...], 0.0)
        o_ref[...] = jnp.dot(h, w4_ref[...], preferred_element_type=jnp.float32) + b4_ref[...]

    def full(arr):
        nd = arr.ndim
        return pl.BlockSpec(arr.shape, lambda i, _nd=nd: tuple(0 for _ in range(_nd)))

    ws = [bc, w1a, w1b, b1, w2, b2, w3, b3, w4p, b4p]
    ws = [v.reshape(1, -1) if v.ndim == 1 else v for v in ws]
    in_specs = [pl.BlockSpec((bm, k), lambda i: (i, 0)),
                pl.BlockSpec((bm, k), lambda i: (i, 0))] + [full(v) for v in ws]
    return pl.pallas_call(
        body,
        grid=(m // bm,),
        in_specs=in_specs,
        out_specs=pl.BlockSpec((bm, 128), lambda i: (i, 0)),
        out_shape=jax.ShapeDtypeStruct((m, 128), jnp.float32),
    )(h0, h1, *ws)


def _sc_edge_scores(xl, xr, att, src, dst, block=80):
    n, f = xl.shape
    e_tot = src.shape[0]
    ew = e_tot // NW
    nchunks = ew // block
    nfc = f // L

    @functools.partial(
        pl.kernel,
        out_type=jax.ShapeDtypeStruct((e_tot,), jnp.float32),
        mesh=_mesh(),
        scratch_types=[
            pltpu.VMEM((block,), jnp.int32),
            pltpu.VMEM((block,), jnp.int32),
            pltpu.VMEM((block, f), jnp.float32),
            pltpu.VMEM((block, f), jnp.float32),
            pltpu.VMEM((f,), jnp.float32),
            pltpu.VMEM((block,), jnp.float32),
        ],
    )
    def k(xl_hbm, xr_hbm, att_hbm, src_hbm, dst_hbm, e_hbm,
          src_v, dst_v, xlr_v, xrr_v, att_v, e_v):
        wid = lax.axis_index("s") * NCORES + lax.axis_index("c")
        base = wid * ew
        pltpu.sync_copy(att_hbm, att_v)
        lane0 = lax.iota(jnp.int32, L) == 0

        def chunk(i, _):
            off = base + i * block
            pltpu.sync_copy(src_hbm.at[pl.ds(off, block)], src_v)
            pltpu.sync_copy(dst_hbm.at[pl.ds(off, block)], dst_v)
            pltpu.sync_copy(xl_hbm.at[src_v], xlr_v)
            pltpu.sync_copy(xr_hbm.at[dst_v], xrr_v)

            def edge(j, _):
                def feat(cc, acc):
                    sl = pl.ds(cc * L, L)
                    v = xlr_v[j, sl] + xrr_v[j, sl]
                    v = jnp.maximum(v, 0.2 * v)
                    return acc + v * att_v[sl]

                acc = lax.fori_loop(0, nfc, feat, jnp.zeros((L,), jnp.float32))
                tot = jnp.sum(acc)
                plsc.store_scatter(e_v, [jnp.full((L,), j, jnp.int32)],
                                   jnp.full((L,), tot, jnp.float32), mask=lane0)
                return 0

            lax.fori_loop(0, block, edge, 0)
            pltpu.sync_copy(e_v, e_hbm.at[pl.ds(off, block)])
            return 0

        lax.fori_loop(0, nchunks, chunk, 0)

    return k(xl, xr, att, src, dst)


def _sc_softmax_agg(xla, xlb, e, src, dst, n_pad, block=80):
    n, f2 = xla.shape
    e_tot = src.shape[0]
    ew = e_tot // NSUB
    nchunks = ew // block
    nfc = f2 // L
    rpt = n_pad // NSUB
    zr = 128
    assert rpt % zr == 0

    @functools.partial(
        pl.kernel,
        out_type=(jax.ShapeDtypeStruct((n_pad, f2), jnp.float32),
                  jax.ShapeDtypeStruct((n_pad, f2), jnp.float32)),
        mesh=_mesh(),
        scratch_types=[
            pltpu.VMEM((block,), jnp.int32),
            pltpu.VMEM((block,), jnp.int32),
            pltpu.VMEM((block,), jnp.float32),
            pltpu.VMEM((ew,), jnp.float32),
            pltpu.VMEM((block,), jnp.float32),
            pltpu.VMEM((block, f2), jnp.float32),
            pltpu.VMEM((n_pad,), jnp.float32),
            pltpu.VMEM((zr, f2), jnp.float32),
            pltpu.VMEM((rpt,), jnp.float32),
            pltpu.VMEM_SHARED((n_pad,), jnp.float32),
            pltpu.VMEM_SHARED((n_pad, f2), jnp.float32),
        ],
    )
    def k(xla_hbm, xlb_hbm, e_hbm, src_hbm, dst_hbm, outa_hbm, outb_hbm,
          src_v, dst_v, e_v, ex_v, alpha_v, rows_v, denom_v, zrows_v, zden_v,
          denom_sh, out_sh):
        c = lax.axis_index("c")
        s = lax.axis_index("s")
        zvec = jnp.zeros((L,), jnp.float32)

        def zfill(i, _):
            zden_v[pl.ds(i * L, L)] = zvec
            return 0

        lax.fori_loop(0, rpt // L, zfill, 0)

        def zfill2(r, _):
            def zf2(cc, _):
                zrows_v[r, pl.ds(cc * L, L)] = zvec
                return 0
            lax.fori_loop(0, nfc, zf2, 0)
            return 0

        lax.fori_loop(0, zr, zfill2, 0)

        pltpu.sync_copy(zden_v, denom_sh.at[pl.ds(s * rpt, rpt)])
        for kk in range(rpt // zr):
            pltpu.sync_copy(zrows_v, out_sh.at[pl.ds(s * rpt + kk * zr, zr)])
        plsc.subcore_barrier()

        base = s * ew

        def chunk1(i, _):
            off = base + i * block
            pltpu.sync_copy(e_hbm.at[pl.ds(off, block)], e_v)
            pltpu.sync_copy(dst_hbm.at[pl.ds(off, block)], dst_v)

            def grp(g, _):
                sl = pl.ds(g * L, L)
                ex_v[pl.ds(i * block + g * L, L)] = jnp.exp(e_v[sl])
                return 0

            lax.fori_loop(0, block // L, grp, 0)
            pltpu.sync_copy(ex_v.at[pl.ds(i * block, block)],
                            denom_sh.at[dst_v], add=True)
            return 0

        lax.fori_loop(0, nchunks, chunk1, 0)
        plsc.subcore_barrier()
        pltpu.sync_copy(denom_sh, denom_v)

        def chunk2(i, _):
            off = base + i * block
            pltpu.sync_copy(src_hbm.at[pl.ds(off, block)], src_v)
            pltpu.sync_copy(dst_hbm.at[pl.ds(off, block)], dst_v)

            @pl.when(c == 0)
            def _():
                pltpu.sync_copy(xla_hbm.at[src_v], rows_v)

            @pl.when(c == 1)
            def _():
                pltpu.sync_copy(xlb_hbm.at[src_v], rows_v)

            def grp(g, _):
                sl = pl.ds(g * L, L)
                d16 = dst_v[sl]
                den = plsc.load_gather(denom_v, [d16])
                alpha_v[sl] = ex_v[pl.ds(i * block + g * L, L)] / (den + 1e-16)
                return 0

            lax.fori_loop(0, block // L, grp, 0)

            def edge(j, _):
                ab = plsc.load_gather(alpha_v, [jnp.full((L,), j, jnp.int32)])

                def feat(cc, _):
                    sl = pl.ds(cc * L, L)
                    rows_v[j, sl] = rows_v[j, sl] * ab
                    return 0

                lax.fori_loop(0, nfc, feat, 0)
                return 0

            lax.fori_loop(0, block, edge, 0)
            pltpu.sync_copy(rows_v, out_sh.at[dst_v], add=True)
            return 0

        lax.fori_loop(0, nchunks, chunk2, 0)
        plsc.subcore_barrier()

        for kk in range(rpt // zr):
            row0 = s * rpt + kk * zr

            @pl.when(c == 0)
            def _():
                pltpu.sync_copy(out_sh.at[pl.ds(row0, zr)],
                                outa_hbm.at[pl.ds(row0, zr)])

            @pl.when(c == 1)
            def _():
                pltpu.sync_copy(out_sh.at[pl.ds(row0, zr)],
                                outb_hbm.at[pl.ds(row0, zr)])

    return k(xla, xlb, e, src, dst)


def _sc_gather_rows(table, idx, block=128):
    n, f = table.shape
    m = idx.shape[0]
    per_w = m // NW
    nchunks = per_w // block

    @functools.partial(
        pl.kernel,
        out_type=jax.ShapeDtypeStruct((m, f), jnp.float32),
        mesh=_mesh(),
        scratch_types=[
            pltpu.VMEM((block,), jnp.int32),
            pltpu.VMEM((block, f), jnp.float32),
        ],
    )
    def k(tab_hbm, idx_hbm, out_hbm, idx_v, rows_v):
        wid = lax.axis_index("s") * NCORES + lax.axis_index("c")
        base = wid * per_w

        def chunk(i, _):
            off = base + i * block
            pltpu.sync_copy(idx_hbm.at[pl.ds(off, block)], idx_v)
            pltpu.sync_copy(tab_hbm.at[idx_v], rows_v)
            pltpu.sync_copy(rows_v, out_hbm.at[pl.ds(off, block)])
            return 0

        lax.fori_loop(0, nchunks, chunk, 0)

    return k(table, idx)


def kernel(x, edge_index, pos_edge_index, neg_edge_index,
           Wl1, Wr1, att1, bc1, Wl2, Wr2, att2, bc2,
           W1, b1, W2, b2, W3, b3, W4, b4):
    n = x.shape[0]
    n_pad = ((n + 2047) // 2048) * 2048
    src = edge_index[0].astype(jnp.int32)
    dst = edge_index[1].astype(jnp.int32)

    out1 = Wl1.shape[1]
    lr1 = _tc_matmul(x, jnp.concatenate([Wl1, Wr1], axis=1))
    xl1 = lr1[:, :out1]
    xr1 = lr1[:, out1:]
    e1 = _sc_edge_scores(xl1, xr1, att1, src, dst)
    h1dim = out1 // 2
    outa1, outb1 = _sc_softmax_agg(xl1[:, :h1dim], xl1[:, h1dim:], e1, src, dst, n_pad)
    agg1 = jnp.concatenate([outa1[:n], outb1[:n]], axis=1)

    out2 = Wl2.shape[1]
    lr2 = _tc_matmul(agg1, jnp.concatenate([Wl2, Wr2], axis=1),
                     pre_bias=bc1, pre_relu=True)
    xl2 = lr2[:, :out2]
    xr2 = lr2[:, out2:]
    e2 = _sc_edge_scores(xl2, xr2, att2, src, dst)
    h2dim = out2 // 2
    outa2, outb2 = _sc_softmax_agg(xl2[:, :h2dim], xl2[:, h2dim:], e2, src, dst, n_pad)
    z2 = jnp.concatenate([outa2[:n], outb2[:n]], axis=1)

    idx_all = jnp.concatenate([pos_edge_index[0], neg_edge_index[0],
                               pos_edge_index[1], neg_edge_index[1]]).astype(jnp.int32)
    rows = _sc_gather_rows(z2, idx_all)
    m = idx_all.shape[0] // 2
    h0 = rows[:m]
    h1 = rows[m:]

    kdim = W1.shape[0] // 2
    w1a = W1[:kdim]
    w1b = W1[kdim:]
    w4p = jnp.pad(W4, ((0, 0), (0, 127)))
    b4p = jnp.pad(b4, (0, 127))
    mlp = _tc_decode_mlp(h0, h1, bc2, w1a, w1b, b1, W2, b2, W3, b3, w4p, b4p)
    return mlp[:, 0]


# speedup vs baseline: 3.5618x; 3.5618x over previous
import functools

import jax
import jax.numpy as jnp
from jax import lax
from jax.experimental import pallas as pl
from jax.experimental.pallas import tpu as pltpu
from jax.experimental.pallas import tpu_sc as plsc

L = 16
NCORES = 2
NSUB = 16
NW = NCORES * NSUB


def _mesh():
    return plsc.VectorSubcoreMesh(core_axis_name="c", subcore_axis_name="s")


def _tc_matmul(a, w, pre_bias=None, pre_relu=False, bm=1000):
    m, k = a.shape
    _, n = w.shape
    assert m % bm == 0

    def body(*refs):
        if pre_bias is None:
            a_ref, w_ref, o_ref = refs
            av = a_ref[...]
        else:
            a_ref, w_ref, b_ref, o_ref = refs
            av = a_ref[...] + b_ref[...]
        if pre_relu:
            av = jnp.maximum(av, 0.0)
        o_ref[...] = jnp.dot(av, w_ref[...], preferred_element_type=jnp.float32)

    in_specs = [
        pl.BlockSpec((bm, k), lambda i: (i, 0)),
        pl.BlockSpec((k, n), lambda i: (0, 0)),
    ]
    args = [a, w]
    if pre_bias is not None:
        in_specs.append(pl.BlockSpec((1, k), lambda i: (0, 0)))
        args.append(pre_bias.reshape(1, k))
    return pl.pallas_call(
        body,
        grid=(m // bm,),
        in_specs=in_specs,
        out_specs=pl.BlockSpec((bm, n), lambda i: (i, 0)),
        out_shape=jax.ShapeDtypeStruct((m, n), jnp.float32),
    )(*args)


def _tc_decode_mlp(h0, h1, bc, w1a, w1b, b1, w2, b2, w3, b3, w4p, b4p, bm=4096):
    m, k = h0.shape

    def body(h0_ref, h1_ref, bc_ref, w1a_ref, w1b_ref, b1_ref, w2_ref, b2_ref,
             w3_ref, b3_ref, w4_ref, b4_ref, o_ref):
        a0 = h0_ref[...] + bc_ref[...]
        a1 = h1_ref[...] + bc_ref[...]
        h = jnp.dot(a0, w1a_ref[...], preferred_element_type=jnp.float32)
        h += jnp.dot(a1, w1b_ref[...], preferred_element_type=jnp.float32)
        h = jnp.maximum(h + b1_ref[...], 0.0)
        h = jnp.maximum(jnp.dot(h, w2_ref[...], preferred_element_type=jnp.float32) + b2_ref[...], 0.0)
        h = jnp.maximum(jnp.dot(h, w3_ref[...], preferred_element_type=jnp.float32) + b3_ref[...], 0.0)
        o_ref[...] = jnp.dot(h, w4_ref[...], preferred_element_type=jnp.float32) + b4_ref[...]

    def full(arr):
        nd = arr.ndim
        return pl.BlockSpec(arr.shape, lambda i, _nd=nd: tuple(0 for _ in range(_nd)))

    ws = [bc, w1a, w1b, b1, w2, b2, w3, b3, w4p, b4p]
    ws = [v.reshape(1, -1) if v.ndim == 1 else v for v in ws]
    in_specs = [pl.BlockSpec((bm, k), lambda i: (i, 0)),
                pl.BlockSpec((bm, k), lambda i: (i, 0))] + [full(v) for v in ws]
    return pl.pallas_call(
        body,
        grid=(m // bm,),
        in_specs=in_specs,
        out_specs=pl.BlockSpec((bm, 128), lambda i: (i, 0)),
        out_shape=jax.ShapeDtypeStruct((m, 128), jnp.float32),
    )(h0, h1, *ws)


def _sc_edge_scores(xl, xr, att, src, dst, block=80):
    n, f = xl.shape
    e_tot = src.shape[0]
    ew = e_tot // NW
    nchunks = ew // block
    nfc = f // L

    @functools.partial(
        pl.kernel,
        out_type=jax.ShapeDtypeStruct((e_tot,), jnp.float32),
        mesh=_mesh(),
        compiler_params=pltpu.CompilerParams(needs_layout_passes=False),
        scratch_types=[
            pltpu.VMEM((block,), jnp.int32),
            pltpu.VMEM((block,), jnp.int32),
            pltpu.VMEM((block, f), jnp.float32),
            pltpu.VMEM((block, f), jnp.float32),
            pltpu.VMEM((f,), jnp.float32),
            pltpu.VMEM((block,), jnp.float32),
            pltpu.VMEM((L * L,), jnp.float32),
        ],
    )
    def k(xl_hbm, xr_hbm, att_hbm, src_hbm, dst_hbm, e_hbm,
          src_v, dst_v, xlr_v, xrr_v, att_v, e_v, m_v):
        wid = lax.axis_index("s") * NCORES + lax.axis_index("c")
        base = wid * ew
        pltpu.sync_copy(att_hbm, att_v)
        rowbase = lax.iota(jnp.int32, L) * L

        def chunk(i, _):
            off = base + i * block
            pltpu.sync_copy(src_hbm.at[pl.ds(off, block)], src_v)
            pltpu.sync_copy(dst_hbm.at[pl.ds(off, block)], dst_v)
            pltpu.sync_copy(xl_hbm.at[src_v], xlr_v)
            pltpu.sync_copy(xr_hbm.at[dst_v], xrr_v)

            def grp(g, _):
                def edge(jj, _):
                    j = g * L + jj

                    def feat(cc, acc):
                        sl = pl.ds(cc * L, L)
                        v = xlr_v[j, sl] + xrr_v[j, sl]
                        v = jnp.maximum(v, 0.2 * v)
                        return acc + v * att_v[sl]

                    acc = lax.fori_loop(0, nfc, feat,
                                        jnp.zeros((L,), jnp.float32))
                    m_v[pl.ds(jj * L, L)] = acc
                    return 0

                lax.fori_loop(0, L, edge, 0)

                def col(cc, tot):
                    return tot + plsc.load_gather(m_v, [rowbase + cc])

                tot16 = lax.fori_loop(0, L, col, jnp.zeros((L,), jnp.float32))
                e_v[pl.ds(g * L, L)] = tot16
                return 0

            lax.fori_loop(0, block // L, grp, 0)
            pltpu.sync_copy(e_v, e_hbm.at[pl.ds(off, block)])
            return 0

        lax.fori_loop(0, nchunks, chunk, 0)

    return k(xl, xr, att, src, dst)


def _sc_softmax_agg(xla, xlb, e, src, dst, n_pad, feature_split, block=80):
    n, f2 = xla.shape
    e_tot = src.shape[0]
    ew = e_tot // NSUB
    nchunks = ew // block
    ew2 = ew if feature_split else ew // NCORES
    nchunks2 = ew2 // block
    nfc = f2 // L
    rpt = n_pad // NSUB
    zr = 32
    cr = 128
    assert rpt % zr == 0 and rpt % cr == 0

    @functools.partial(
        pl.kernel,
        out_type=(jax.ShapeDtypeStruct((n_pad, f2), jnp.float32),
                  jax.ShapeDtypeStruct((n_pad, f2), jnp.float32)),
        mesh=_mesh(),
        compiler_params=pltpu.CompilerParams(needs_layout_passes=False),
        scratch_types=[
            pltpu.VMEM((block,), jnp.int32),
            pltpu.VMEM((block,), jnp.int32),
            pltpu.VMEM((block,), jnp.float32),
            pltpu.VMEM((block,), jnp.float32),
            pltpu.VMEM((block,), jnp.float32),
            pltpu.VMEM((block, f2), jnp.float32),
            pltpu.VMEM((n_pad,), jnp.float32),
            pltpu.VMEM((zr, f2), jnp.float32),
            pltpu.VMEM((rpt,), jnp.float32),
            pltpu.VMEM_SHARED((n_pad,), jnp.float32),
            pltpu.VMEM_SHARED((n_pad, f2), jnp.float32),
        ],
    )
    def k(xla_hbm, xlb_hbm, e_hbm, src_hbm, dst_hbm, outa_hbm, outb_hbm,
          src_v, dst_v, e_v, exb_v, alpha_v, rows_v, denom_v, zrows_v, zden_v,
          denom_sh, out_sh):
        c = lax.axis_index("c")
        s = lax.axis_index("s")
        zvec = jnp.zeros((L,), jnp.float32)

        def zfill(i, _):
            zden_v[pl.ds(i * L, L)] = zvec
            return 0

        lax.fori_loop(0, rpt // L, zfill, 0)

        def zfill2(r, _):
            def zf2(cc, _):
                zrows_v[r, pl.ds(cc * L, L)] = zvec
                return 0
            lax.fori_loop(0, nfc, zf2, 0)
            return 0

        lax.fori_loop(0, zr, zfill2, 0)

        pltpu.sync_copy(zden_v, denom_sh.at[pl.ds(s * rpt, rpt)])
        for kk in range(rpt // zr):
            pltpu.sync_copy(zrows_v, out_sh.at[pl.ds(s * rpt + kk * zr, zr)])
        plsc.subcore_barrier()

        base = s * ew

        def chunk1(i, _):
            off = base + i * block
            pltpu.sync_copy(e_hbm.at[pl.ds(off, block)], e_v)
            pltpu.sync_copy(dst_hbm.at[pl.ds(off, block)], dst_v)

            def grp(g, _):
                sl = pl.ds(g * L, L)
                exb_v[sl] = jnp.exp(e_v[sl])
                return 0

            lax.fori_loop(0, block // L, grp, 0)
            pltpu.sync_copy(exb_v, denom_sh.at[dst_v], add=True)
            return 0

        lax.fori_loop(0, nchunks, chunk1, 0)
        plsc.subcore_barrier()
        pltpu.sync_copy(denom_sh, denom_v)

        if feature_split:
            base2 = s * ew2
        else:
            base2 = (c * NSUB + s) * ew2

        def chunk2(i, _):
            off = base2 + i * block
            pltpu.sync_copy(src_hbm.at[pl.ds(off, block)], src_v)
            pltpu.sync_copy(dst_hbm.at[pl.ds(off, block)], dst_v)
            pltpu.sync_copy(e_hbm.at[pl.ds(off, block)], e_v)

            if feature_split:
                @pl.when(c == 0)
                def _():
                    pltpu.sync_copy(xla_hbm.at[src_v], rows_v)

                @pl.when(c == 1)
                def _():
                    pltpu.sync_copy(xlb_hbm.at[src_v], rows_v)
            else:
                pltpu.sync_copy(xla_hbm.at[src_v], rows_v)

            def grp(g, _):
                sl = pl.ds(g * L, L)
                d16 = dst_v[sl]
                den = plsc.load_gather(denom_v, [d16])
                alpha_v[sl] = jnp.exp(e_v[sl]) / (den + 1e-16)
                return 0

            lax.fori_loop(0, block // L, grp, 0)

            def edge(j, _):
                ab = plsc.load_gather(alpha_v, [jnp.full((L,), j, jnp.int32)])

                def feat(cc, _):
                    sl = pl.ds(cc * L, L)
                    rows_v[j, sl] = rows_v[j, sl] * ab
                    return 0

                lax.fori_loop(0, nfc, feat, 0)
                return 0

            lax.fori_loop(0, block, edge, 0)
            pltpu.sync_copy(rows_v, out_sh.at[dst_v], add=True)
            return 0

        lax.fori_loop(0, nchunks2, chunk2, 0)
        plsc.subcore_barrier()

        for kk in range(rpt // cr):
            row0 = s * rpt + kk * cr

            @pl.when(c == 0)
            def _():
                pltpu.sync_copy(out_sh.at[pl.ds(row0, cr)],
                                outa_hbm.at[pl.ds(row0, cr)])

            @pl.when(c == 1)
            def _():
                pltpu.sync_copy(out_sh.at[pl.ds(row0, cr)],
                                outb_hbm.at[pl.ds(row0, cr)])

    return k(xla, xlb, e, src, dst)


def _sc_gather_rows_sum2(table_a, table_b, idx, block=128):
    n, f = table_a.shape
    m = idx.shape[0]
    per_w = m // NW
    nchunks = per_w // block
    nfc = f // L

    @functools.partial(
        pl.kernel,
        out_type=jax.ShapeDtypeStruct((m, f), jnp.float32),
        mesh=_mesh(),
        compiler_params=pltpu.CompilerParams(needs_layout_passes=False),
        scratch_types=[
            pltpu.VMEM((block,), jnp.int32),
            pltpu.VMEM((block, f), jnp.float32),
            pltpu.VMEM((block, f), jnp.float32),
        ],
    )
    def k(taba_hbm, tabb_hbm, idx_hbm, out_hbm, idx_v, rowsa_v, rowsb_v):
        wid = lax.axis_index("s") * NCORES + lax.axis_index("c")
        base = wid * per_w

        def chunk(i, _):
            off = base + i * block
            pltpu.sync_copy(idx_hbm.at[pl.ds(off, block)], idx_v)
            pltpu.sync_copy(taba_hbm.at[idx_v], rowsa_v)
            pltpu.sync_copy(tabb_hbm.at[idx_v], rowsb_v)

            def row(j, _):
                def feat(cc, _):
                    sl = pl.ds(cc * L, L)
                    rowsa_v[j, sl] = rowsa_v[j, sl] + rowsb_v[j, sl]
                    return 0

                lax.fori_loop(0, nfc, feat, 0)
                return 0

            lax.fori_loop(0, block, row, 0)
            pltpu.sync_copy(rowsa_v, out_hbm.at[pl.ds(off, block)])
            return 0

        lax.fori_loop(0, nchunks, chunk, 0)

    return k(table_a, table_b, idx)


def kernel(x, edge_index, pos_edge_index, neg_edge_index,
           Wl1, Wr1, att1, bc1, Wl2, Wr2, att2, bc2,
           W1, b1, W2, b2, W3, b3, W4, b4):
    n = x.shape[0]
    n_pad = ((n + 2047) // 2048) * 2048
    src = edge_index[0].astype(jnp.int32)
    dst = edge_index[1].astype(jnp.int32)

    out1 = Wl1.shape[1]
    lr1 = _tc_matmul(x, jnp.concatenate([Wl1, Wr1], axis=1))
    xl1 = lr1[:, :out1]
    xr1 = lr1[:, out1:]
    e1 = _sc_edge_scores(xl1, xr1, att1, src, dst)
    h1dim = out1 // 2
    outa1, outb1 = _sc_softmax_agg(xl1[:, :h1dim], xl1[:, h1dim:], e1, src, dst,
                                   n_pad, feature_split=True)
    agg1 = jnp.concatenate([outa1[:n], outb1[:n]], axis=1)

    out2 = Wl2.shape[1]
    lr2 = _tc_matmul(agg1, jnp.concatenate([Wl2, Wr2], axis=1),
                     pre_bias=bc1, pre_relu=True)
    xl2 = lr2[:, :out2]
    xr2 = lr2[:, out2:]
    e2 = _sc_edge_scores(xl2, xr2, att2, src, dst)
    outa2, outb2 = _sc_softmax_agg(xl2, xl2, e2, src, dst,
                                   n_pad, feature_split=False)

    idx_all = jnp.concatenate([pos_edge_index[0], neg_edge_index[0],
                               pos_edge_index[1], neg_edge_index[1]]).astype(jnp.int32)
    rows = _sc_gather_rows_sum2(outa2, outb2, idx_all)
    m = idx_all.shape[0] // 2
    h0 = rows[:m]
    h1 = rows[m:]

    kdim = W1.shape[0] // 2
    w1a = W1[:kdim]
    w1b = W1[kdim:]
    w4p = jnp.pad(W4, ((0, 0), (0, 127)))
    b4p = jnp.pad(b4, (0, 127))
    mlp = _tc_decode_mlp(h0, h1, bc2, w1a, w1b, b1, W2, b2, W3, b3, w4p, b4p)
    return mlp[:, 0]


# speedup vs baseline: 4.0892x; 1.1480x over previous
import functools

import jax
import jax.numpy as jnp
from jax import lax
from jax.experimental import pallas as pl
from jax.experimental.pallas import tpu as pltpu
from jax.experimental.pallas import tpu_sc as plsc

L = 16
NCORES = 2
NSUB = 16
NW = NCORES * NSUB


def _mesh():
    return plsc.VectorSubcoreMesh(core_axis_name="c", subcore_axis_name="s")


def _tc_matmul(a, w, pre_bias=None, pre_relu=False, bm=1000):
    m, k = a.shape
    _, n = w.shape
    assert m % bm == 0

    def body(*refs):
        if pre_bias is None:
            a_ref, w_ref, o_ref = refs
            av = a_ref[...]
        else:
            a_ref, w_ref, b_ref, o_ref = refs
            av = a_ref[...] + b_ref[...]
        if pre_relu:
            av = jnp.maximum(av, 0.0)
        o_ref[...] = jnp.dot(av, w_ref[...], preferred_element_type=jnp.float32)

    in_specs = [
        pl.BlockSpec((bm, k), lambda i: (i, 0)),
        pl.BlockSpec((k, n), lambda i: (0, 0)),
    ]
    args = [a, w]
    if pre_bias is not None:
        in_specs.append(pl.BlockSpec((1, k), lambda i: (0, 0)))
        args.append(pre_bias.reshape(1, k))
    return pl.pallas_call(
        body,
        grid=(m // bm,),
        in_specs=in_specs,
        out_specs=pl.BlockSpec((bm, n), lambda i: (i, 0)),
        out_shape=jax.ShapeDtypeStruct((m, n), jnp.float32),
    )(*args)


def _tc_decode_mlp(h0, h1, bc, w1a, w1b, b1, w2, b2, w3, b3, w4p, b4p, bm=4096):
    m, k = h0.shape

    def body(h0_ref, h1_ref, bc_ref, w1a_ref, w1b_ref, b1_ref, w2_ref, b2_ref,
             w3_ref, b3_ref, w4_ref, b4_ref, o_ref):
        a0 = h0_ref[...] + bc_ref[...]
        a1 = h1_ref[...] + bc_ref[...]
        h = jnp.dot(a0, w1a_ref[...], preferred_element_type=jnp.float32)
        h += jnp.dot(a1, w1b_ref[...], preferred_element_type=jnp.float32)
        h = jnp.maximum(h + b1_ref[...], 0.0)
        h = jnp.maximum(jnp.dot(h, w2_ref[...], preferred_element_type=jnp.float32) + b2_ref[...], 0.0)
        h = jnp.maximum(jnp.dot(h, w3_ref[...], preferred_element_type=jnp.float32) + b3_ref[...], 0.0)
        o_ref[...] = jnp.dot(h, w4_ref[...], preferred_element_type=jnp.float32) + b4_ref[...]

    def full(arr):
        nd = arr.ndim
        return pl.BlockSpec(arr.shape, lambda i, _nd=nd: tuple(0 for _ in range(_nd)))

    ws = [bc, w1a, w1b, b1, w2, b2, w3, b3, w4p, b4p]
    ws = [v.reshape(1, -1) if v.ndim == 1 else v for v in ws]
    in_specs = [pl.BlockSpec((bm, k), lambda i: (i, 0)),
                pl.BlockSpec((bm, k), lambda i: (i, 0))] + [full(v) for v in ws]
    return pl.pallas_call(
        body,
        grid=(m // bm,),
        in_specs=in_specs,
        out_specs=pl.BlockSpec((bm, 128), lambda i: (i, 0)),
        out_shape=jax.ShapeDtypeStruct((m, 128), jnp.float32),
    )(h0, h1, *ws)


def _sc_edge_scores(xl, xr, att, src, dst, block=80):
    n, f = xl.shape
    e_tot = src.shape[0]
    ew = e_tot // NW
    nchunks = ew // block
    nfc = f // L

    @functools.partial(
        pl.kernel,
        out_type=jax.ShapeDtypeStruct((e_tot,), jnp.float32),
        mesh=_mesh(),
        compiler_params=pltpu.CompilerParams(needs_layout_passes=False),
        scratch_types=[
            pltpu.VMEM((block,), jnp.int32),
            pltpu.VMEM((block,), jnp.int32),
            pltpu.VMEM((block, f), jnp.float32),
            pltpu.VMEM((block, f), jnp.float32),
            pltpu.VMEM((f,), jnp.float32),
            pltpu.VMEM((block,), jnp.float32),
            pltpu.VMEM((L * L,), jnp.float32),
        ],
    )
    def k(xl_hbm, xr_hbm, att_hbm, src_hbm, dst_hbm, e_hbm,
          src_v, dst_v, xlr_v, xrr_v, att_v, e_v, m_v):
        wid = lax.axis_index("s") * NCORES + lax.axis_index("c")
        base = wid * ew
        pltpu.sync_copy(att_hbm, att_v)
        rowbase = lax.iota(jnp.int32, L) * L

        def chunk(i, _):
            off = base + i * block
            pltpu.sync_copy(src_hbm.at[pl.ds(off, block)], src_v)
            pltpu.sync_copy(dst_hbm.at[pl.ds(off, block)], dst_v)
            pltpu.sync_copy(xl_hbm.at[src_v], xlr_v)
            pltpu.sync_copy(xr_hbm.at[dst_v], xrr_v)

            def grp(g, _):
                for jj in range(L):
                    j = g * L + jj
                    accs = [jnp.zeros((L,), jnp.float32) for _ in range(4)]
                    for cc in range(nfc):
                        sl = pl.ds(cc * L, L)
                        v = xlr_v[j, sl] + xrr_v[j, sl]
                        v = jnp.maximum(v, 0.2 * v)
                        accs[cc % 4] = accs[cc % 4] + v * att_v[sl]
                    m_v[pl.ds(jj * L, L)] = (accs[0] + accs[1]) + (accs[2] + accs[3])

                cols = [plsc.load_gather(m_v, [rowbase + cc]) for cc in range(L)]
                while len(cols) > 1:
                    cols = [cols[t] + cols[t + 1] for t in range(0, len(cols), 2)]
                e_v[pl.ds(g * L, L)] = cols[0]
                return 0

            lax.fori_loop(0, block // L, grp, 0)
            pltpu.sync_copy(e_v, e_hbm.at[pl.ds(off, block)])
            return 0

        lax.fori_loop(0, nchunks, chunk, 0)

    return k(xl, xr, att, src, dst)


def _sc_softmax_agg(xla, xlb, e, src, dst, n_pad, feature_split, block=80):
    n, f2 = xla.shape
    e_tot = src.shape[0]
    ew = e_tot // NSUB
    nchunks = ew // block
    ew2 = ew if feature_split else ew // NCORES
    nchunks2 = ew2 // block
    nfc = f2 // L
    rpt = n_pad // NSUB
    zr = 32
    cr = 128
    assert rpt % zr == 0 and rpt % cr == 0

    @functools.partial(
        pl.kernel,
        out_type=(jax.ShapeDtypeStruct((n_pad, f2), jnp.float32),
                  jax.ShapeDtypeStruct((n_pad, f2), jnp.float32)),
        mesh=_mesh(),
        compiler_params=pltpu.CompilerParams(needs_layout_passes=False),
        scratch_types=[
            pltpu.VMEM((block,), jnp.int32),
            pltpu.VMEM((block,), jnp.int32),
            pltpu.VMEM((block,), jnp.float32),
            pltpu.VMEM((block,), jnp.float32),
            pltpu.VMEM((block,), jnp.float32),
            pltpu.VMEM((block, f2), jnp.float32),
            pltpu.VMEM((n_pad,), jnp.float32),
            pltpu.VMEM((zr, f2), jnp.float32),
            pltpu.VMEM((rpt,), jnp.float32),
            pltpu.VMEM_SHARED((n_pad,), jnp.float32),
            pltpu.VMEM_SHARED((n_pad, f2), jnp.float32),
        ],
    )
    def k(xla_hbm, xlb_hbm, e_hbm, src_hbm, dst_hbm, outa_hbm, outb_hbm,
          src_v, dst_v, e_v, exb_v, alpha_v, rows_v, denom_v, zrows_v, zden_v,
          denom_sh, out_sh):
        c = lax.axis_index("c")
        s = lax.axis_index("s")
        zvec = jnp.zeros((L,), jnp.float32)

        def zfill(i, _):
            zden_v[pl.ds(i * L, L)] = zvec
            return 0

        lax.fori_loop(0, rpt // L, zfill, 0)

        def zfill2(r, _):
            def zf2(cc, _):
                zrows_v[r, pl.ds(cc * L, L)] = zvec
                return 0
            lax.fori_loop(0, nfc, zf2, 0)
            return 0

        lax.fori_loop(0, zr, zfill2, 0)

        pltpu.sync_copy(zden_v, denom_sh.at[pl.ds(s * rpt, rpt)])
        for kk in range(rpt // zr):
            pltpu.sync_copy(zrows_v, out_sh.at[pl.ds(s * rpt + kk * zr, zr)])
        plsc.subcore_barrier()

        base = s * ew

        def chunk1(i, _):
            off = base + i * block
            pltpu.sync_copy(e_hbm.at[pl.ds(off, block)], e_v)
            pltpu.sync_copy(dst_hbm.at[pl.ds(off, block)], dst_v)

            def grp(g, _):
                sl = pl.ds(g * L, L)
                exb_v[sl] = jnp.exp(e_v[sl])
                return 0

            lax.fori_loop(0, block // L, grp, 0)
            pltpu.sync_copy(exb_v, denom_sh.at[dst_v], add=True)
            return 0

        lax.fori_loop(0, nchunks, chunk1, 0)
        plsc.subcore_barrier()
        pltpu.sync_copy(denom_sh, denom_v)

        if feature_split:
            base2 = s * ew2
        else:
            base2 = (c * NSUB + s) * ew2

        def chunk2(i, _):
            off = base2 + i * block
            pltpu.sync_copy(src_hbm.at[pl.ds(off, block)], src_v)
            pltpu.sync_copy(dst_hbm.at[pl.ds(off, block)], dst_v)
            pltpu.sync_copy(e_hbm.at[pl.ds(off, block)], e_v)

            if feature_split:
                @pl.when(c == 0)
                def _():
                    pltpu.sync_copy(xla_hbm.at[src_v], rows_v)

                @pl.when(c == 1)
                def _():
                    pltpu.sync_copy(xlb_hbm.at[src_v], rows_v)
            else:
                pltpu.sync_copy(xla_hbm.at[src_v], rows_v)

            def grp(g, _):
                sl = pl.ds(g * L, L)
                d16 = dst_v[sl]
                den = plsc.load_gather(denom_v, [d16])
                alpha_v[sl] = jnp.exp(e_v[sl]) / (den + 1e-16)
                return 0

            lax.fori_loop(0, block // L, grp, 0)

            def edge_grp(gg, _):
                for jj in range(L):
                    j = gg * L + jj
                    ab = plsc.load_gather(alpha_v, [jnp.full((L,), j, jnp.int32)])
                    for cc in range(nfc):
                        sl = pl.ds(cc * L, L)
                        rows_v[j, sl] = rows_v[j, sl] * ab
                return 0

            lax.fori_loop(0, block // L, edge_grp, 0)
            pltpu.sync_copy(rows_v, out_sh.at[dst_v], add=True)
            return 0

        lax.fori_loop(0, nchunks2, chunk2, 0)
        plsc.subcore_barrier()

        for kk in range(rpt // cr):
            row0 = s * rpt + kk * cr

            @pl.when(c == 0)
            def _():
                pltpu.sync_copy(out_sh.at[pl.ds(row0, cr)],
                                outa_hbm.at[pl.ds(row0, cr)])

            @pl.when(c == 1)
            def _():
                pltpu.sync_copy(out_sh.at[pl.ds(row0, cr)],
                                outb_hbm.at[pl.ds(row0, cr)])

    return k(xla, xlb, e, src, dst)


def _sc_gather_rows_sum2(table_a, table_b, idx, block=128):
    n, f = table_a.shape
    m = idx.shape[0]
    per_w = m // NW
    nchunks = per_w // block
    nfc = f // L

    @functools.partial(
        pl.kernel,
        out_type=jax.ShapeDtypeStruct((m, f), jnp.float32),
        mesh=_mesh(),
        compiler_params=pltpu.CompilerParams(needs_layout_passes=False),
        scratch_types=[
            pltpu.VMEM((block,), jnp.int32),
            pltpu.VMEM((block, f), jnp.float32),
            pltpu.VMEM((block, f), jnp.float32),
        ],
    )
    def k(taba_hbm, tabb_hbm, idx_hbm, out_hbm, idx_v, rowsa_v, rowsb_v):
        wid = lax.axis_index("s") * NCORES + lax.axis_index("c")
        base = wid * per_w

        def chunk(i, _):
            off = base + i * block
            pltpu.sync_copy(idx_hbm.at[pl.ds(off, block)], idx_v)
            pltpu.sync_copy(taba_hbm.at[idx_v], rowsa_v)
            pltpu.sync_copy(tabb_hbm.at[idx_v], rowsb_v)

            def row_grp(gg, _):
                for jj in range(8):
                    j = gg * 8 + jj
                    for cc in range(nfc):
                        sl = pl.ds(cc * L, L)
                        rowsa_v[j, sl] = rowsa_v[j, sl] + rowsb_v[j, sl]
                return 0

            lax.fori_loop(0, block // 8, row_grp, 0)
            pltpu.sync_copy(rowsa_v, out_hbm.at[pl.ds(off, block)])
            return 0

        lax.fori_loop(0, nchunks, chunk, 0)

    return k(table_a, table_b, idx)


def kernel(x, edge_index, pos_edge_index, neg_edge_index,
           Wl1, Wr1, att1, bc1, Wl2, Wr2, att2, bc2,
           W1, b1, W2, b2, W3, b3, W4, b4):
    n = x.shape[0]
    n_pad = ((n + 2047) // 2048) * 2048
    src = edge_index[0].astype(jnp.int32)
    dst = edge_index[1].astype(jnp.int32)

    out1 = Wl1.shape[1]
    lr1 = _tc_matmul(x, jnp.concatenate([Wl1, Wr1], axis=1))
    xl1 = lr1[:, :out1]
    xr1 = lr1[:, out1:]
    e1 = _sc_edge_scores(xl1, xr1, att1, src, dst)
    h1dim = out1 // 2
    outa1, outb1 = _sc_softmax_agg(xl1[:, :h1dim], xl1[:, h1dim:], e1, src, dst,
                                   n_pad, feature_split=True)
    agg1 = jnp.concatenate([outa1[:n], outb1[:n]], axis=1)

    out2 = Wl2.shape[1]
    lr2 = _tc_matmul(agg1, jnp.concatenate([Wl2, Wr2], axis=1),
                     pre_bias=bc1, pre_relu=True)
    xl2 = lr2[:, :out2]
    xr2 = lr2[:, out2:]
    e2 = _sc_edge_scores(xl2, xr2, att2, src, dst)
    outa2, outb2 = _sc_softmax_agg(xl2, xl2, e2, src, dst,
                                   n_pad, feature_split=False)

    idx_all = jnp.concatenate([pos_edge_index[0], neg_edge_index[0],
                               pos_edge_index[1], neg_edge_index[1]]).astype(jnp.int32)
    rows = _sc_gather_rows_sum2(outa2, outb2, idx_all)
    m = idx_all.shape[0] // 2
    h0 = rows[:m]
    h1 = rows[m:]

    kdim = W1.shape[0] // 2
    w1a = W1[:kdim]
    w1b = W1[kdim:]
    w4p = jnp.pad(W4, ((0, 0), (0, 127)))
    b4p = jnp.pad(b4, (0, 127))
    mlp = _tc_decode_mlp(h0, h1, bc2, w1a, w1b, b1, W2, b2, W3, b3, w4p, b4p)
    return mlp[:, 0]


# speedup vs baseline: 5.5218x; 1.3504x over previous
import functools

import jax
import jax.numpy as jnp
from jax import lax
from jax.experimental import pallas as pl
from jax.experimental.pallas import tpu as pltpu
from jax.experimental.pallas import tpu_sc as plsc

L = 16
NCORES = 2
NSUB = 16
NW = NCORES * NSUB


def _mesh():
    return plsc.VectorSubcoreMesh(core_axis_name="c", subcore_axis_name="s")


def _tc_matmul(a, w, pre_bias=None, pre_relu=False, bm=1000):
    m, k = a.shape
    _, n = w.shape
    assert m % bm == 0

    def body(*refs):
        if pre_bias is None:
            a_ref, w_ref, o_ref = refs
            av = a_ref[...]
        else:
            a_ref, w_ref, b_ref, o_ref = refs
            av = a_ref[...] + b_ref[...]
        if pre_relu:
            av = jnp.maximum(av, 0.0)
        o_ref[...] = jnp.dot(av, w_ref[...], preferred_element_type=jnp.float32)

    in_specs = [
        pl.BlockSpec((bm, k), lambda i: (i, 0)),
        pl.BlockSpec((k, n), lambda i: (0, 0)),
    ]
    args = [a, w]
    if pre_bias is not None:
        in_specs.append(pl.BlockSpec((1, k), lambda i: (0, 0)))
        args.append(pre_bias.reshape(1, k))
    return pl.pallas_call(
        body,
        grid=(m // bm,),
        in_specs=in_specs,
        out_specs=pl.BlockSpec((bm, n), lambda i: (i, 0)),
        out_shape=jax.ShapeDtypeStruct((m, n), jnp.float32),
    )(*args)


def _tc_decode_mlp(h0, h1, bc, w1a, w1b, b1, w2, b2, w3, b3, w4p, b4p, bm=4096):
    m, k = h0.shape

    def body(h0_ref, h1_ref, bc_ref, w1a_ref, w1b_ref, b1_ref, w2_ref, b2_ref,
             w3_ref, b3_ref, w4_ref, b4_ref, o_ref):
        a0 = h0_ref[...] + bc_ref[...]
        a1 = h1_ref[...] + bc_ref[...]
        h = jnp.dot(a0, w1a_ref[...], preferred_element_type=jnp.float32)
        h += jnp.dot(a1, w1b_ref[...], preferred_element_type=jnp.float32)
        h = jnp.maximum(h + b1_ref[...], 0.0)
        h = jnp.maximum(jnp.dot(h, w2_ref[...], preferred_element_type=jnp.float32) + b2_ref[...], 0.0)
        h = jnp.maximum(jnp.dot(h, w3_ref[...], preferred_element_type=jnp.float32) + b3_ref[...], 0.0)
        o_ref[...] = jnp.dot(h, w4_ref[...], preferred_element_type=jnp.float32) + b4_ref[...]

    def full(arr):
        nd = arr.ndim
        return pl.BlockSpec(arr.shape, lambda i, _nd=nd: tuple(0 for _ in range(_nd)))

    ws = [bc, w1a, w1b, b1, w2, b2, w3, b3, w4p, b4p]
    ws = [v.reshape(1, -1) if v.ndim == 1 else v for v in ws]
    in_specs = [pl.BlockSpec((bm, k), lambda i: (i, 0)),
                pl.BlockSpec((bm, k), lambda i: (i, 0))] + [full(v) for v in ws]
    return pl.pallas_call(
        body,
        grid=(m // bm,),
        in_specs=in_specs,
        out_specs=pl.BlockSpec((bm, 128), lambda i: (i, 0)),
        out_shape=jax.ShapeDtypeStruct((m, 128), jnp.float32),
    )(h0, h1, *ws)


def _sc_edge_scores(xl, xr, att, src, dst, block=80):
    n, f = xl.shape
    e_tot = src.shape[0]
    ew = e_tot // NW
    nchunks = ew // block
    nfc = f // L

    @functools.partial(
        pl.kernel,
        out_type=jax.ShapeDtypeStruct((e_tot,), jnp.float32),
        mesh=_mesh(),
        compiler_params=pltpu.CompilerParams(needs_layout_passes=False),
        scratch_types=[
            pltpu.VMEM((block,), jnp.int32),
            pltpu.VMEM((block,), jnp.int32),
            pltpu.VMEM((block,), jnp.int32),
            pltpu.VMEM((block,), jnp.int32),
            pltpu.VMEM((block, f), jnp.float32),
            pltpu.VMEM((block, f), jnp.float32),
            pltpu.VMEM((block, f), jnp.float32),
            pltpu.VMEM((block, f), jnp.float32),
            pltpu.VMEM((f,), jnp.float32),
            pltpu.VMEM((block,), jnp.float32),
            pltpu.VMEM((L * L,), jnp.float32),
            pltpu.SemaphoreType.DMA,
            pltpu.SemaphoreType.DMA,
        ],
    )
    def k(xl_hbm, xr_hbm, att_hbm, src_hbm, dst_hbm, e_hbm,
          src0_v, src1_v, dst0_v, dst1_v, xlr0_v, xlr1_v, xrr0_v, xrr1_v,
          att_v, e_v, m_v, sem0, sem1):
        wid = lax.axis_index("s") * NCORES + lax.axis_index("c")
        base = wid * ew
        pltpu.sync_copy(att_hbm, att_v)
        rowbase = lax.iota(jnp.int32, L) * L
        slots = ((src0_v, dst0_v, xlr0_v, xrr0_v, sem0),
                 (src1_v, dst1_v, xlr1_v, xrr1_v, sem1))

        def issue(i, slot):
            sv, dv, xlv, xrv, sem = slots[slot]
            off = base + i * block
            pltpu.sync_copy(src_hbm.at[pl.ds(off, block)], sv)
            pltpu.sync_copy(dst_hbm.at[pl.ds(off, block)], dv)
            pltpu.async_copy(xl_hbm.at[sv], xlv, sem)
            pltpu.async_copy(xr_hbm.at[dv], xrv, sem)

        def wait(slot):
            sv, dv, xlv, xrv, sem = slots[slot]
            pltpu.make_async_copy(xl_hbm.at[sv], xlv, sem).wait()
            pltpu.make_async_copy(xr_hbm.at[dv], xrv, sem).wait()

        def compute(i, slot):
            _, _, xlr_v, xrr_v, _ = slots[slot]
            off = base + i * block

            def grp(g, _):
                for jj in range(L):
                    j = g * L + jj
                    accs = [jnp.zeros((L,), jnp.float32) for _ in range(4)]
                    for cc in range(nfc):
                        sl = pl.ds(cc * L, L)
                        v = xlr_v[j, sl] + xrr_v[j, sl]
                        v = jnp.maximum(v, 0.2 * v)
                        accs[cc % 4] = accs[cc % 4] + v * att_v[sl]
                    m_v[pl.ds(jj * L, L)] = (accs[0] + accs[1]) + (accs[2] + accs[3])

                cols = [plsc.load_gather(m_v, [rowbase + cc]) for cc in range(L)]
                while len(cols) > 1:
                    cols = [cols[t] + cols[t + 1] for t in range(0, len(cols), 2)]
                e_v[pl.ds(g * L, L)] = cols[0]
                return 0

            lax.fori_loop(0, block // L, grp, 0)
            pltpu.sync_copy(e_v, e_hbm.at[pl.ds(off, block)])

        assert nchunks % 2 == 1

        def pair(t, _):
            i0 = 2 * t
            issue(i0 + 1, 1)
            wait(0)
            compute(i0, 0)
            issue(i0 + 2, 0)
            wait(1)
            compute(i0 + 1, 1)
            return 0

        issue(0, 0)
        lax.fori_loop(0, (nchunks - 1) // 2, pair, 0)
        wait(0)
        compute(nchunks - 1, 0)

    return k(xl, xr, att, src, dst)


def _sc_softmax_agg(xla, xlb, e, src, dst, n_pad, feature_split, block=80):
    n, f2 = xla.shape
    e_tot = src.shape[0]
    ew = e_tot // NSUB
    nchunks = ew // block
    ew2 = ew if feature_split else ew // NCORES
    nchunks2 = ew2 // block
    nfc = f2 // L
    rpt = n_pad // NSUB
    zr = 32
    cr = 128
    assert rpt % zr == 0 and rpt % cr == 0

    @functools.partial(
        pl.kernel,
        out_type=(jax.ShapeDtypeStruct((n_pad, f2), jnp.float32),
                  jax.ShapeDtypeStruct((n_pad, f2), jnp.float32)),
        mesh=_mesh(),
        compiler_params=pltpu.CompilerParams(needs_layout_passes=False),
        scratch_types=[
            pltpu.VMEM((block,), jnp.int32),
            pltpu.VMEM((block,), jnp.int32),
            pltpu.VMEM((block,), jnp.int32),
            pltpu.VMEM((block,), jnp.int32),
            pltpu.VMEM((block,), jnp.float32),
            pltpu.VMEM((block,), jnp.float32),
            pltpu.VMEM((block,), jnp.float32),
            pltpu.VMEM((block, f2), jnp.float32),
            pltpu.VMEM((block, f2), jnp.float32),
            pltpu.VMEM((n_pad,), jnp.float32),
            pltpu.VMEM((zr, f2), jnp.float32),
            pltpu.VMEM((rpt,), jnp.float32),
            pltpu.VMEM_SHARED((n_pad,), jnp.float32),
            pltpu.VMEM_SHARED((n_pad, f2), jnp.float32),
            pltpu.SemaphoreType.DMA,
            pltpu.SemaphoreType.DMA,
        ],
    )
    def k(xla_hbm, xlb_hbm, e_hbm, src_hbm, dst_hbm, outa_hbm, outb_hbm,
          src0_v, src1_v, dst0_v, dst1_v, e_v, exb_v, alpha_v,
          rows0_v, rows1_v, denom_v, zrows_v, zden_v,
          denom_sh, out_sh, sem0, sem1):
        c = lax.axis_index("c")
        s = lax.axis_index("s")
        zvec = jnp.zeros((L,), jnp.float32)

        def zfill(i, _):
            zden_v[pl.ds(i * L, L)] = zvec
            return 0

        lax.fori_loop(0, rpt // L, zfill, 0)

        def zfill2(r, _):
            def zf2(cc, _):
                zrows_v[r, pl.ds(cc * L, L)] = zvec
                return 0
            lax.fori_loop(0, nfc, zf2, 0)
            return 0

        lax.fori_loop(0, zr, zfill2, 0)

        pltpu.sync_copy(zden_v, denom_sh.at[pl.ds(s * rpt, rpt)])
        for kk in range(rpt // zr):
            pltpu.sync_copy(zrows_v, out_sh.at[pl.ds(s * rpt + kk * zr, zr)])
        plsc.subcore_barrier()

        base = s * ew

        def chunk1(i, _):
            off = base + i * block
            pltpu.sync_copy(e_hbm.at[pl.ds(off, block)], e_v)
            pltpu.sync_copy(dst_hbm.at[pl.ds(off, block)], dst0_v)

            def grp(g, _):
                sl = pl.ds(g * L, L)
                exb_v[sl] = jnp.exp(e_v[sl])
                return 0

            lax.fori_loop(0, block // L, grp, 0)
            pltpu.sync_copy(exb_v, denom_sh.at[dst0_v], add=True)
            return 0

        lax.fori_loop(0, nchunks, chunk1, 0)
        plsc.subcore_barrier()
        pltpu.sync_copy(denom_sh, denom_v)

        if feature_split:
            base2 = s * ew2
        else:
            base2 = (c * NSUB + s) * ew2

        slots = ((src0_v, dst0_v, rows0_v, sem0),
                 (src1_v, dst1_v, rows1_v, sem1))

        def issue(i, slot):
            sv, dv, rv, sem = slots[slot]
            off = base2 + i * block
            pltpu.sync_copy(src_hbm.at[pl.ds(off, block)], sv)
            pltpu.sync_copy(dst_hbm.at[pl.ds(off, block)], dv)
            if feature_split:
                @pl.when(c == 0)
                def _():
                    pltpu.async_copy(xla_hbm.at[sv], rv, sem)

                @pl.when(c == 1)
                def _():
                    pltpu.async_copy(xlb_hbm.at[sv], rv, sem)
            else:
                pltpu.async_copy(xla_hbm.at[sv], rv, sem)

        def wait(slot):
            sv, dv, rv, sem = slots[slot]
            pltpu.make_async_copy(xla_hbm.at[sv], rv, sem).wait()

        def compute(i, slot):
            sv, dst_v, rows_v, sem = slots[slot]
            off = base2 + i * block
            pltpu.sync_copy(e_hbm.at[pl.ds(off, block)], e_v)

            def grp(g, _):
                sl = pl.ds(g * L, L)
                d16 = dst_v[sl]
                den = plsc.load_gather(denom_v, [d16])
                alpha_v[sl] = jnp.exp(e_v[sl]) / (den + 1e-16)
                return 0

            lax.fori_loop(0, block // L, grp, 0)
            wait(slot)

            def edge_grp(gg, _):
                for jj in range(L):
                    j = gg * L + jj
                    ab = plsc.load_gather(alpha_v, [jnp.full((L,), j, jnp.int32)])
                    for cc in range(nfc):
                        sl = pl.ds(cc * L, L)
                        rows_v[j, sl] = rows_v[j, sl] * ab
                return 0

            lax.fori_loop(0, block // L, edge_grp, 0)
            pltpu.sync_copy(rows_v, out_sh.at[dst_v], add=True)

        issue(0, 0)
        if nchunks2 % 2 == 1:
            def pair(t, _):
                i0 = 2 * t
                issue(i0 + 1, 1)
                compute(i0, 0)
                issue(i0 + 2, 0)
                compute(i0 + 1, 1)
                return 0

            lax.fori_loop(0, (nchunks2 - 1) // 2, pair, 0)
            compute(nchunks2 - 1, 0)
        else:
            def pair(t, _):
                i0 = 2 * t
                issue(i0 + 1, 1)
                compute(i0, 0)
                issue(i0 + 2, 0)
                compute(i0 + 1, 1)
                return 0

            lax.fori_loop(0, nchunks2 // 2 - 1, pair, 0)
            i0 = nchunks2 - 2
            issue(i0 + 1, 1)
            compute(i0, 0)
            compute(i0 + 1, 1)
        plsc.subcore_barrier()

        for kk in range(rpt // cr):
            row0 = s * rpt + kk * cr

            @pl.when(c == 0)
            def _():
                pltpu.sync_copy(out_sh.at[pl.ds(row0, cr)],
                                outa_hbm.at[pl.ds(row0, cr)])

            @pl.when(c == 1)
            def _():
                pltpu.sync_copy(out_sh.at[pl.ds(row0, cr)],
                                outb_hbm.at[pl.ds(row0, cr)])

    return k(xla, xlb, e, src, dst)


def _sc_gather_rows_sum2(table_a, table_b, idx, block=128):
    n, f = table_a.shape
    m = idx.shape[0]
    per_w = m // NW
    nchunks = per_w // block
    nfc = f // L

    @functools.partial(
        pl.kernel,
        out_type=jax.ShapeDtypeStruct((m, f), jnp.float32),
        mesh=_mesh(),
        compiler_params=pltpu.CompilerParams(needs_layout_passes=False),
        scratch_types=[
            pltpu.VMEM((block,), jnp.int32),
            pltpu.VMEM((block, f), jnp.float32),
            pltpu.VMEM((block, f), jnp.float32),
        ],
    )
    def k(taba_hbm, tabb_hbm, idx_hbm, out_hbm, idx_v, rowsa_v, rowsb_v):
        wid = lax.axis_index("s") * NCORES + lax.axis_index("c")
        base = wid * per_w

        def chunk(i, _):
            off = base + i * block
            pltpu.sync_copy(idx_hbm.at[pl.ds(off, block)], idx_v)
            pltpu.sync_copy(taba_hbm.at[idx_v], rowsa_v)
            pltpu.sync_copy(tabb_hbm.at[idx_v], rowsb_v)

            def row_grp(gg, _):
                for jj in range(8):
                    j = gg * 8 + jj
                    for cc in range(nfc):
                        sl = pl.ds(cc * L, L)
                        rowsa_v[j, sl] = rowsa_v[j, sl] + rowsb_v[j, sl]
                return 0

            lax.fori_loop(0, block // 8, row_grp, 0)
            pltpu.sync_copy(rowsa_v, out_hbm.at[pl.ds(off, block)])
            return 0

        lax.fori_loop(0, nchunks, chunk, 0)

    return k(table_a, table_b, idx)


def kernel(x, edge_index, pos_edge_index, neg_edge_index,
           Wl1, Wr1, att1, bc1, Wl2, Wr2, att2, bc2,
           W1, b1, W2, b2, W3, b3, W4, b4):
    n = x.shape[0]
    n_pad = ((n + 2047) // 2048) * 2048
    src = edge_index[0].astype(jnp.int32)
    dst = edge_index[1].astype(jnp.int32)

    out1 = Wl1.shape[1]
    lr1 = _tc_matmul(x, jnp.concatenate([Wl1, Wr1], axis=1))
    xl1 = lr1[:, :out1]
    xr1 = lr1[:, out1:]
    e1 = _sc_edge_scores(xl1, xr1, att1, src, dst)
    h1dim = out1 // 2
    outa1, outb1 = _sc_softmax_agg(xl1[:, :h1dim], xl1[:, h1dim:], e1, src, dst,
                                   n_pad, feature_split=True)
    agg1 = jnp.concatenate([outa1[:n], outb1[:n]], axis=1)

    out2 = Wl2.shape[1]
    lr2 = _tc_matmul(agg1, jnp.concatenate([Wl2, Wr2], axis=1),
                     pre_bias=bc1, pre_relu=True)
    xl2 = lr2[:, :out2]
    xr2 = lr2[:, out2:]
    e2 = _sc_edge_scores(xl2, xr2, att2, src, dst)
    outa2, outb2 = _sc_softmax_agg(xl2, xl2, e2, src, dst,
                                   n_pad, feature_split=False)

    idx_all = jnp.concatenate([pos_edge_index[0], neg_edge_index[0],
                               pos_edge_index[1], neg_edge_index[1]]).astype(jnp.int32)
    rows = _sc_gather_rows_sum2(outa2, outb2, idx_all)
    m = idx_all.shape[0] // 2
    h0 = rows[:m]
    h1 = rows[m:]

    kdim = W1.shape[0] // 2
    w1a = W1[:kdim]
    w1b = W1[kdim:]
    w4p = jnp.pad(W4, ((0, 0), (0, 127)))
    b4p = jnp.pad(b4, (0, 127))
    mlp = _tc_decode_mlp(h0, h1, bc2, w1a, w1b, b1, W2, b2, W3, b3, w4p, b4p)
    return mlp[:, 0]


# speedup vs baseline: 5.8618x; 1.0616x over previous
import functools

import jax
import jax.numpy as jnp
from jax import lax
from jax.experimental import pallas as pl
from jax.experimental.pallas import tpu as pltpu
from jax.experimental.pallas import tpu_sc as plsc

L = 16
NCORES = 2
NSUB = 16
NW = NCORES * NSUB


def _mesh():
    return plsc.VectorSubcoreMesh(core_axis_name="c", subcore_axis_name="s")


def _tc_matmul(a, w, pre_bias=None, pre_relu=False, bm=1000):
    m, k = a.shape
    _, n = w.shape
    assert m % bm == 0

    def body(*refs):
        if pre_bias is None:
            a_ref, w_ref, o_ref = refs
            av = a_ref[...]
        else:
            a_ref, w_ref, b_ref, o_ref = refs
            av = a_ref[...] + b_ref[...]
        if pre_relu:
            av = jnp.maximum(av, 0.0)
        o_ref[...] = jnp.dot(av, w_ref[...], preferred_element_type=jnp.float32)

    in_specs = [
        pl.BlockSpec((bm, k), lambda i: (i, 0)),
        pl.BlockSpec((k, n), lambda i: (0, 0)),
    ]
    args = [a, w]
    if pre_bias is not None:
        in_specs.append(pl.BlockSpec((1, k), lambda i: (0, 0)))
        args.append(pre_bias.reshape(1, k))
    return pl.pallas_call(
        body,
        grid=(m // bm,),
        in_specs=in_specs,
        out_specs=pl.BlockSpec((bm, n), lambda i: (i, 0)),
        out_shape=jax.ShapeDtypeStruct((m, n), jnp.float32),
    )(*args)


def _tc_decode_mlp(h0, h1, bc, w1a, w1b, b1, w2, b2, w3, b3, w4p, b4p, bm=4096):
    m, k = h0.shape

    def body(h0_ref, h1_ref, bc_ref, w1a_ref, w1b_ref, b1_ref, w2_ref, b2_ref,
             w3_ref, b3_ref, w4_ref, b4_ref, o_ref):
        a0 = h0_ref[...] + bc_ref[...]
        a1 = h1_ref[...] + bc_ref[...]
        h = jnp.dot(a0, w1a_ref[...], preferred_element_type=jnp.float32)
        h += jnp.dot(a1, w1b_ref[...], preferred_element_type=jnp.float32)
        h = jnp.maximum(h + b1_ref[...], 0.0)
        h = jnp.maximum(jnp.dot(h, w2_ref[...], preferred_element_type=jnp.float32) + b2_ref[...], 0.0)
        h = jnp.maximum(jnp.dot(h, w3_ref[...], preferred_element_type=jnp.float32) + b3_ref[...], 0.0)
        o_ref[...] = jnp.dot(h, w4_ref[...], preferred_element_type=jnp.float32) + b4_ref[...]

    def full(arr):
        nd = arr.ndim
        return pl.BlockSpec(arr.shape, lambda i, _nd=nd: tuple(0 for _ in range(_nd)))

    ws = [bc, w1a, w1b, b1, w2, b2, w3, b3, w4p, b4p]
    ws = [v.reshape(1, -1) if v.ndim == 1 else v for v in ws]
    in_specs = [pl.BlockSpec((bm, k), lambda i: (i, 0)),
                pl.BlockSpec((bm, k), lambda i: (i, 0))] + [full(v) for v in ws]
    return pl.pallas_call(
        body,
        grid=(m // bm,),
        in_specs=in_specs,
        out_specs=pl.BlockSpec((bm, 128), lambda i: (i, 0)),
        out_shape=jax.ShapeDtypeStruct((m, 128), jnp.float32),
    )(h0, h1, *ws)


def _sc_edge_scores(xl, xr, att, src, dst, block=80):
    n, f = xl.shape
    e_tot = src.shape[0]
    ew = e_tot // NW
    nchunks = ew // block
    nfc = f // L

    @functools.partial(
        pl.kernel,
        out_type=jax.ShapeDtypeStruct((e_tot,), jnp.float32),
        mesh=_mesh(),
        compiler_params=pltpu.CompilerParams(needs_layout_passes=False),
        scratch_types=[
            pltpu.VMEM((block,), jnp.int32),
            pltpu.VMEM((block,), jnp.int32),
            pltpu.VMEM((block,), jnp.int32),
            pltpu.VMEM((block,), jnp.int32),
            pltpu.VMEM((block, f), jnp.float32),
            pltpu.VMEM((block, f), jnp.float32),
            pltpu.VMEM((block, f), jnp.float32),
            pltpu.VMEM((block, f), jnp.float32),
            pltpu.VMEM((f,), jnp.float32),
            pltpu.VMEM((block,), jnp.float32),
            pltpu.VMEM((L * L,), jnp.float32),
            pltpu.SemaphoreType.DMA,
            pltpu.SemaphoreType.DMA,
        ],
    )
    def k(xl_hbm, xr_hbm, att_hbm, src_hbm, dst_hbm, e_hbm,
          src0_v, src1_v, dst0_v, dst1_v, xlr0_v, xlr1_v, xrr0_v, xrr1_v,
          att_v, e_v, m_v, sem0, sem1):
        wid = lax.axis_index("s") * NCORES + lax.axis_index("c")
        base = wid * ew
        pltpu.sync_copy(att_hbm, att_v)
        rowbase = lax.iota(jnp.int32, L) * L
        slots = ((src0_v, dst0_v, xlr0_v, xrr0_v, sem0),
                 (src1_v, dst1_v, xlr1_v, xrr1_v, sem1))

        def issue(i, slot):
            sv, dv, xlv, xrv, sem = slots[slot]
            off = base + i * block
            pltpu.sync_copy(src_hbm.at[pl.ds(off, block)], sv)
            pltpu.sync_copy(dst_hbm.at[pl.ds(off, block)], dv)
            pltpu.async_copy(xl_hbm.at[sv], xlv, sem)
            pltpu.async_copy(xr_hbm.at[dv], xrv, sem)

        def wait(slot):
            sv, dv, xlv, xrv, sem = slots[slot]
            pltpu.make_async_copy(xl_hbm.at[sv], xlv, sem).wait()
            pltpu.make_async_copy(xr_hbm.at[dv], xrv, sem).wait()

        def compute(i, slot):
            _, _, xlr_v, xrr_v, _ = slots[slot]
            off = base + i * block

            def grp(g, _):
                for jj in range(L):
                    j = g * L + jj
                    accs = [jnp.zeros((L,), jnp.float32) for _ in range(4)]
                    for cc in range(nfc):
                        sl = pl.ds(cc * L, L)
                        v = xlr_v[j, sl] + xrr_v[j, sl]
                        v = jnp.maximum(v, 0.2 * v)
                        accs[cc % 4] = accs[cc % 4] + v * att_v[sl]
                    m_v[pl.ds(jj * L, L)] = (accs[0] + accs[1]) + (accs[2] + accs[3])

                cols = [plsc.load_gather(m_v, [rowbase + cc]) for cc in range(L)]
                while len(cols) > 1:
                    cols = [cols[t] + cols[t + 1] for t in range(0, len(cols), 2)]
                e_v[pl.ds(g * L, L)] = cols[0]
                return 0

            lax.fori_loop(0, block // L, grp, 0)
            pltpu.sync_copy(e_v, e_hbm.at[pl.ds(off, block)])

        assert nchunks % 2 == 1

        def pair(t, _):
            i0 = 2 * t
            issue(i0 + 1, 1)
            wait(0)
            compute(i0, 0)
            issue(i0 + 2, 0)
            wait(1)
            compute(i0 + 1, 1)
            return 0

        issue(0, 0)
        lax.fori_loop(0, (nchunks - 1) // 2, pair, 0)
        wait(0)
        compute(nchunks - 1, 0)

    return k(xl, xr, att, src, dst)


def _sc_softmax_agg(xla, xlb, e, src, dst, n_pad, feature_split, block=80):
    n, f2 = xla.shape
    e_tot = src.shape[0]
    ew = e_tot // NSUB
    nchunks = ew // block
    ew2 = ew if feature_split else ew // NCORES
    nchunks2 = ew2 // block
    nfc = f2 // L
    rpt = n_pad // NSUB
    cr = 128
    assert rpt % block == 0 and rpt % cr == 0

    @functools.partial(
        pl.kernel,
        out_type=(jax.ShapeDtypeStruct((n_pad, f2), jnp.float32),
                  jax.ShapeDtypeStruct((n_pad, f2), jnp.float32)),
        mesh=_mesh(),
        compiler_params=pltpu.CompilerParams(needs_layout_passes=False),
        scratch_types=[
            pltpu.VMEM((block,), jnp.int32),
            pltpu.VMEM((block,), jnp.int32),
            pltpu.VMEM((block,), jnp.int32),
            pltpu.VMEM((block,), jnp.int32),
            pltpu.VMEM((block,), jnp.int32),
            pltpu.VMEM((block,), jnp.float32),
            pltpu.VMEM((block,), jnp.float32),
            pltpu.VMEM((block,), jnp.float32),
            pltpu.VMEM((block,), jnp.float32),
            pltpu.VMEM((block, f2), jnp.float32),
            pltpu.VMEM((block, f2), jnp.float32),
            pltpu.VMEM((block, f2), jnp.float32),
            pltpu.VMEM((n_pad,), jnp.float32),
            pltpu.VMEM((rpt,), jnp.float32),
            pltpu.VMEM_SHARED((n_pad,), jnp.float32),
            pltpu.VMEM_SHARED((n_pad, f2), jnp.float32),
            pltpu.SemaphoreType.DMA,
            pltpu.SemaphoreType.DMA,
            pltpu.SemaphoreType.DMA,
        ],
    )
    def k(xla_hbm, xlb_hbm, e_hbm, src_hbm, dst_hbm, outa_hbm, outb_hbm,
          src0_v, src1_v, dst0_v, dst1_v, sdst_v,
          e_v, exb0_v, exb1_v, alpha_v,
          rows0_v, rows1_v, srows_v, denom_v, zden_v,
          denom_sh, out_sh, gsem0, gsem1, ssem):
        c = lax.axis_index("c")
        s = lax.axis_index("s")
        zvec = jnp.zeros((L,), jnp.float32)

        def zfill(i, _):
            zden_v[pl.ds(i * L, L)] = zvec
            return 0

        lax.fori_loop(0, rpt // L, zfill, 0)

        def zfill2(r, _):
            def zf2(cc, _):
                srows_v[r, pl.ds(cc * L, L)] = zvec
                return 0
            lax.fori_loop(0, nfc, zf2, 0)
            return 0

        lax.fori_loop(0, block, zfill2, 0)

        pltpu.sync_copy(zden_v, denom_sh.at[pl.ds(s * rpt, rpt)])
        for kk in range(rpt // block):
            pltpu.sync_copy(srows_v, out_sh.at[pl.ds(s * rpt + kk * block, block)])
        plsc.subcore_barrier()

        base = s * ew

        def chunk1(i, _):
            off = base + i * block
            pltpu.sync_copy(e_hbm.at[pl.ds(off, block)], e_v)
            pltpu.sync_copy(dst_hbm.at[pl.ds(off, block)], dst0_v)

            def grp(g, _):
                sl = pl.ds(g * L, L)
                exb0_v[sl] = jnp.exp(e_v[sl])
                return 0

            lax.fori_loop(0, block // L, grp, 0)
            pltpu.sync_copy(exb0_v, denom_sh.at[dst0_v], add=True)
            return 0

        lax.fori_loop(0, nchunks, chunk1, 0)
        plsc.subcore_barrier()
        pltpu.sync_copy(denom_sh, denom_v)

        if feature_split:
            base2 = s * ew2
        else:
            base2 = (c * NSUB + s) * ew2

        slots = ((src0_v, dst0_v, rows0_v, exb0_v, gsem0),
                 (src1_v, dst1_v, rows1_v, exb1_v, gsem1))

        def issue2(i, slot):
            sv, dv, rv, ev2, gsem = slots[slot]

            def do():
                off = base2 + i * block
                pltpu.sync_copy(src_hbm.at[pl.ds(off, block)], sv)
                pltpu.sync_copy(dst_hbm.at[pl.ds(off, block)], dv)
                pltpu.sync_copy(e_hbm.at[pl.ds(off, block)], ev2)
                if feature_split:
                    @pl.when(c == 0)
                    def _():
                        pltpu.async_copy(xla_hbm.at[sv], rv, gsem)

                    @pl.when(c == 1)
                    def _():
                        pltpu.async_copy(xlb_hbm.at[sv], rv, gsem)
                else:
                    pltpu.async_copy(xla_hbm.at[sv], rv, gsem)

            if isinstance(i, int):
                if i < nchunks2:
                    do()
            else:
                pl.when(i < nchunks2)(do)

        def phase2(i, slot, first):
            sv, dv, rv, ev2, gsem = slots[slot]
            pltpu.make_async_copy(xla_hbm.at[sv], rv, gsem).wait()

            def grp(g, _):
                sl = pl.ds(g * L, L)
                den = plsc.load_gather(denom_v, [dv[sl]])
                alpha_v[sl] = jnp.exp(ev2[sl]) / (den + 1e-16)
                return 0

            lax.fori_loop(0, block // L, grp, 0)
            if not first:
                pltpu.make_async_copy(srows_v, out_sh.at[sdst_v], ssem).wait()

            def edge_grp(gg, _):
                for jj in range(L):
                    j = gg * L + jj
                    ab = plsc.load_gather(alpha_v, [jnp.full((L,), j, jnp.int32)])
                    for cc in range(nfc):
                        sl = pl.ds(cc * L, L)
                        srows_v[j, sl] = rv[j, sl] * ab
                return 0

            lax.fori_loop(0, block // L, edge_grp, 0)

            def icp(g, _):
                sl = pl.ds(g * L, L)
                sdst_v[sl] = dv[sl]
                return 0

            lax.fori_loop(0, block // L, icp, 0)
            pltpu.async_copy(srows_v, out_sh.at[sdst_v], ssem, add=True)
            issue2(i + 2, slot)

        issue2(0, 0)
        issue2(1, 1)
        phase2(0, 0, True)
        phase2(1, 1, False)

        def pair2(t, _):
            phase2(2 * t, 0, False)
            phase2(2 * t + 1, 1, False)
            return 0

        if nchunks2 % 2 == 0:
            lax.fori_loop(1, nchunks2 // 2, pair2, 0)
        else:
            lax.fori_loop(1, (nchunks2 - 1) // 2, pair2, 0)
            phase2(nchunks2 - 1, 0, False)
        pltpu.make_async_copy(srows_v, out_sh.at[sdst_v], ssem).wait()
        plsc.subcore_barrier()

        for kk in range(rpt // cr):
            row0 = s * rpt + kk * cr

            @pl.when(c == 0)
            def _():
                pltpu.sync_copy(out_sh.at[pl.ds(row0, cr)],
                                outa_hbm.at[pl.ds(row0, cr)])

            @pl.when(c == 1)
            def _():
                pltpu.sync_copy(out_sh.at[pl.ds(row0, cr)],
                                outb_hbm.at[pl.ds(row0, cr)])

    return k(xla, xlb, e, src, dst)


def _sc_gather_rows_sum2(table_a, table_b, idx, block=128):
    n, f = table_a.shape
    m = idx.shape[0]
    per_w = m // NW
    nchunks = per_w // block
    nfc = f // L

    @functools.partial(
        pl.kernel,
        out_type=jax.ShapeDtypeStruct((m, f), jnp.float32),
        mesh=_mesh(),
        compiler_params=pltpu.CompilerParams(needs_layout_passes=False),
        scratch_types=[
            pltpu.VMEM((block,), jnp.int32),
            pltpu.VMEM((block, f), jnp.float32),
            pltpu.VMEM((block, f), jnp.float32),
        ],
    )
    def k(taba_hbm, tabb_hbm, idx_hbm, out_hbm, idx_v, rowsa_v, rowsb_v):
        wid = lax.axis_index("s") * NCORES + lax.axis_index("c")
        base = wid * per_w

        def chunk(i, _):
            off = base + i * block
            pltpu.sync_copy(idx_hbm.at[pl.ds(off, block)], idx_v)
            pltpu.sync_copy(taba_hbm.at[idx_v], rowsa_v)
            pltpu.sync_copy(tabb_hbm.at[idx_v], rowsb_v)

            def row_grp(gg, _):
                for jj in range(8):
                    j = gg * 8 + jj
                    for cc in range(nfc):
                        sl = pl.ds(cc * L, L)
                        rowsa_v[j, sl] = rowsa_v[j, sl] + rowsb_v[j, sl]
                return 0

            lax.fori_loop(0, block // 8, row_grp, 0)
            pltpu.sync_copy(rowsa_v, out_hbm.at[pl.ds(off, block)])
            return 0

        lax.fori_loop(0, nchunks, chunk, 0)

    return k(table_a, table_b, idx)


def kernel(x, edge_index, pos_edge_index, neg_edge_index,
           Wl1, Wr1, att1, bc1, Wl2, Wr2, att2, bc2,
           W1, b1, W2, b2, W3, b3, W4, b4):
    n = x.shape[0]
    n_pad = ((n + 2047) // 2048) * 2048
    src = edge_index[0].astype(jnp.int32)
    dst = edge_index[1].astype(jnp.int32)

    out1 = Wl1.shape[1]
    lr1 = _tc_matmul(x, jnp.concatenate([Wl1, Wr1], axis=1))
    xl1 = lr1[:, :out1]
    xr1 = lr1[:, out1:]
    e1 = _sc_edge_scores(xl1, xr1, att1, src, dst)
    h1dim = out1 // 2
    outa1, outb1 = _sc_softmax_agg(xl1[:, :h1dim], xl1[:, h1dim:], e1, src, dst,
                                   n_pad, feature_split=True)
    agg1 = jnp.concatenate([outa1[:n], outb1[:n]], axis=1)

    out2 = Wl2.shape[1]
    lr2 = _tc_matmul(agg1, jnp.concatenate([Wl2, Wr2], axis=1),
                     pre_bias=bc1, pre_relu=True)
    xl2 = lr2[:, :out2]
    xr2 = lr2[:, out2:]
    e2 = _sc_edge_scores(xl2, xr2, att2, src, dst)
    outa2, outb2 = _sc_softmax_agg(xl2, xl2, e2, src, dst,
                                   n_pad, feature_split=False)

    idx_all = jnp.concatenate([pos_edge_index[0], neg_edge_index[0],
                               pos_edge_index[1], neg_edge_index[1]]).astype(jnp.int32)
    rows = _sc_gather_rows_sum2(outa2, outb2, idx_all)
    m = idx_all.shape[0] // 2
    h0 = rows[:m]
    h1 = rows[m:]

    kdim = W1.shape[0] // 2
    w1a = W1[:kdim]
    w1b = W1[kdim:]
    w4p = jnp.pad(W4, ((0, 0), (0, 127)))
    b4p = jnp.pad(b4, (0, 127))
    mlp = _tc_decode_mlp(h0, h1, bc2, w1a, w1b, b1, W2, b2, W3, b3, w4p, b4p)
    return mlp[:, 0]


# speedup vs baseline: 6.1507x; 1.0493x over previous
import functools

import jax
import jax.numpy as jnp
from jax import lax
from jax.experimental import pallas as pl
from jax.experimental.pallas import tpu as pltpu
from jax.experimental.pallas import tpu_sc as plsc

L = 16
NCORES = 2
NSUB = 16
NW = NCORES * NSUB


def _mesh():
    return plsc.VectorSubcoreMesh(core_axis_name="c", subcore_axis_name="s")


def _tc_matmul(a, w, pre_bias=None, pre_relu=False, bm=1000):
    m, k = a.shape
    _, n = w.shape
    assert m % bm == 0

    def body(*refs):
        if pre_bias is None:
            a_ref, w_ref, o_ref = refs
            av = a_ref[...]
        else:
            a_ref, w_ref, b_ref, o_ref = refs
            av = a_ref[...] + b_ref[...]
        if pre_relu:
            av = jnp.maximum(av, 0.0)
        o_ref[...] = jnp.dot(av, w_ref[...], preferred_element_type=jnp.float32)

    in_specs = [
        pl.BlockSpec((bm, k), lambda i: (i, 0)),
        pl.BlockSpec((k, n), lambda i: (0, 0)),
    ]
    args = [a, w]
    if pre_bias is not None:
        in_specs.append(pl.BlockSpec((1, k), lambda i: (0, 0)))
        args.append(pre_bias.reshape(1, k))
    return pl.pallas_call(
        body,
        grid=(m // bm,),
        in_specs=in_specs,
        out_specs=pl.BlockSpec((bm, n), lambda i: (i, 0)),
        out_shape=jax.ShapeDtypeStruct((m, n), jnp.float32),
    )(*args)


def _tc_decode_mlp(h0, h1, bc, w1a, w1b, b1, w2, b2, w3, b3, w4p, b4p, bm=4096):
    m, k = h0.shape

    def body(h0_ref, h1_ref, bc_ref, w1a_ref, w1b_ref, b1_ref, w2_ref, b2_ref,
             w3_ref, b3_ref, w4_ref, b4_ref, o_ref):
        a0 = h0_ref[...] + bc_ref[...]
        a1 = h1_ref[...] + bc_ref[...]
        h = jnp.dot(a0, w1a_ref[...], preferred_element_type=jnp.float32)
        h += jnp.dot(a1, w1b_ref[...], preferred_element_type=jnp.float32)
        h = jnp.maximum(h + b1_ref[...], 0.0)
        h = jnp.maximum(jnp.dot(h, w2_ref[...], preferred_element_type=jnp.float32) + b2_ref[...], 0.0)
        h = jnp.maximum(jnp.dot(h, w3_ref[...], preferred_element_type=jnp.float32) + b3_ref[...], 0.0)
        o_ref[...] = jnp.dot(h, w4_ref[...], preferred_element_type=jnp.float32) + b4_ref[...]

    def full(arr):
        nd = arr.ndim
        return pl.BlockSpec(arr.shape, lambda i, _nd=nd: tuple(0 for _ in range(_nd)))

    ws = [bc, w1a, w1b, b1, w2, b2, w3, b3, w4p, b4p]
    ws = [v.reshape(1, -1) if v.ndim == 1 else v for v in ws]
    in_specs = [pl.BlockSpec((bm, k), lambda i: (i, 0)),
                pl.BlockSpec((bm, k), lambda i: (i, 0))] + [full(v) for v in ws]
    return pl.pallas_call(
        body,
        grid=(m // bm,),
        in_specs=in_specs,
        out_specs=pl.BlockSpec((bm, 128), lambda i: (i, 0)),
        out_shape=jax.ShapeDtypeStruct((m, 128), jnp.float32),
    )(h0, h1, *ws)


def _sc_edge_scores(xl, xr, att, src, dst, block=80):
    n, f = xl.shape
    e_tot = src.shape[0]
    ew = e_tot // NW
    nchunks = ew // block
    nfc = f // L

    @functools.partial(
        pl.kernel,
        out_type=jax.ShapeDtypeStruct((e_tot,), jnp.float32),
        mesh=_mesh(),
        compiler_params=pltpu.CompilerParams(needs_layout_passes=False),
        scratch_types=[
            pltpu.VMEM((block,), jnp.int32),
            pltpu.VMEM((block,), jnp.int32),
            pltpu.VMEM((block,), jnp.int32),
            pltpu.VMEM((block,), jnp.int32),
            pltpu.VMEM((block, f), jnp.float32),
            pltpu.VMEM((block, f), jnp.float32),
            pltpu.VMEM((block, f), jnp.float32),
            pltpu.VMEM((block, f), jnp.float32),
            pltpu.VMEM((f,), jnp.float32),
            pltpu.VMEM((block,), jnp.float32),
            pltpu.VMEM((L * L,), jnp.float32),
            pltpu.SemaphoreType.DMA,
            pltpu.SemaphoreType.DMA,
        ],
    )
    def k(xl_hbm, xr_hbm, att_hbm, src_hbm, dst_hbm, e_hbm,
          src0_v, src1_v, dst0_v, dst1_v, xlr0_v, xlr1_v, xrr0_v, xrr1_v,
          att_v, e_v, m_v, sem0, sem1):
        wid = lax.axis_index("s") * NCORES + lax.axis_index("c")
        base = wid * ew
        pltpu.sync_copy(att_hbm, att_v)
        rowbase = lax.iota(jnp.int32, L) * L
        slots = ((src0_v, dst0_v, xlr0_v, xrr0_v, sem0),
                 (src1_v, dst1_v, xlr1_v, xrr1_v, sem1))

        def issue(i, slot):
            sv, dv, xlv, xrv, sem = slots[slot]
            off = base + i * block
            pltpu.sync_copy(src_hbm.at[pl.ds(off, block)], sv)
            pltpu.sync_copy(dst_hbm.at[pl.ds(off, block)], dv)
            pltpu.async_copy(xl_hbm.at[sv], xlv, sem)
            pltpu.async_copy(xr_hbm.at[dv], xrv, sem)

        def wait(slot):
            sv, dv, xlv, xrv, sem = slots[slot]
            pltpu.make_async_copy(xl_hbm.at[sv], xlv, sem).wait()
            pltpu.make_async_copy(xr_hbm.at[dv], xrv, sem).wait()

        def compute(i, slot):
            _, _, xlr_v, xrr_v, _ = slots[slot]
            off = base + i * block

            def grp(g, _):
                atts = [att_v[pl.ds(cc * L, L)] for cc in range(nfc)]
                for jj in range(L):
                    j = g * L + jj
                    accs = [jnp.zeros((L,), jnp.float32) for _ in range(4)]
                    for cc in range(nfc):
                        sl = pl.ds(cc * L, L)
                        v = xlr_v[j, sl] + xrr_v[j, sl]
                        v = jnp.maximum(v, 0.2 * v)
                        accs[cc % 4] = accs[cc % 4] + v * atts[cc]
                    m_v[pl.ds(jj * L, L)] = (accs[0] + accs[1]) + (accs[2] + accs[3])

                cols = [plsc.load_gather(m_v, [rowbase + cc]) for cc in range(L)]
                while len(cols) > 1:
                    cols = [cols[t] + cols[t + 1] for t in range(0, len(cols), 2)]
                e_v[pl.ds(g * L, L)] = cols[0]
                return 0

            lax.fori_loop(0, block // L, grp, 0)
            pltpu.sync_copy(e_v, e_hbm.at[pl.ds(off, block)])

        assert nchunks % 2 == 1

        def pair(t, _):
            i0 = 2 * t
            issue(i0 + 1, 1)
            wait(0)
            compute(i0, 0)
            issue(i0 + 2, 0)
            wait(1)
            compute(i0 + 1, 1)
            return 0

        issue(0, 0)
        lax.fori_loop(0, (nchunks - 1) // 2, pair, 0)
        wait(0)
        compute(nchunks - 1, 0)

    return k(xl, xr, att, src, dst)


def _sc_softmax_agg(xla, xlb, e, src, dst, n_pad, feature_split, block=80):
    n, f2 = xla.shape
    e_tot = src.shape[0]
    ew = e_tot // NSUB
    nchunks = ew // block
    ew2 = ew if feature_split else ew // NCORES
    nchunks2 = ew2 // block
    nfc = f2 // L
    rpt = n_pad // NSUB
    cr = 128
    assert rpt % block == 0 and rpt % cr == 0

    @functools.partial(
        pl.kernel,
        out_type=(jax.ShapeDtypeStruct((n_pad, f2), jnp.float32),
                  jax.ShapeDtypeStruct((n_pad, f2), jnp.float32)),
        mesh=_mesh(),
        compiler_params=pltpu.CompilerParams(needs_layout_passes=False),
        scratch_types=[
            pltpu.VMEM((block,), jnp.int32),
            pltpu.VMEM((block,), jnp.int32),
            pltpu.VMEM((block,), jnp.int32),
            pltpu.VMEM((block,), jnp.int32),
            pltpu.VMEM((block,), jnp.int32),
            pltpu.VMEM((block,), jnp.float32),
            pltpu.VMEM((block,), jnp.float32),
            pltpu.VMEM((block,), jnp.float32),
            pltpu.VMEM((block,), jnp.float32),
            pltpu.VMEM((block, f2), jnp.float32),
            pltpu.VMEM((block, f2), jnp.float32),
            pltpu.VMEM((block, f2), jnp.float32),
            pltpu.VMEM((n_pad,), jnp.float32),
            pltpu.VMEM((rpt,), jnp.float32),
            pltpu.VMEM_SHARED((n_pad,), jnp.float32),
            pltpu.VMEM_SHARED((n_pad, f2), jnp.float32),
            pltpu.SemaphoreType.DMA,
            pltpu.SemaphoreType.DMA,
            pltpu.SemaphoreType.DMA,
        ],
    )
    def k(xla_hbm, xlb_hbm, e_hbm, src_hbm, dst_hbm, outa_hbm, outb_hbm,
          src0_v, src1_v, dst0_v, dst1_v, sdst_v,
          e_v, exb0_v, exb1_v, alpha_v,
          rows0_v, rows1_v, srows_v, denom_v, zden_v,
          denom_sh, out_sh, gsem0, gsem1, ssem):
        c = lax.axis_index("c")
        s = lax.axis_index("s")
        zvec = jnp.zeros((L,), jnp.float32)

        def zfill(i, _):
            zden_v[pl.ds(i * L, L)] = zvec
            return 0

        lax.fori_loop(0, rpt // L, zfill, 0)

        def zfill2(r, _):
            def zf2(cc, _):
                srows_v[r, pl.ds(cc * L, L)] = zvec
                return 0
            lax.fori_loop(0, nfc, zf2, 0)
            return 0

        lax.fori_loop(0, block, zfill2, 0)

        pltpu.sync_copy(zden_v, denom_sh.at[pl.ds(s * rpt, rpt)])
        for kk in range(rpt // block):
            pltpu.sync_copy(srows_v, out_sh.at[pl.ds(s * rpt + kk * block, block)])
        plsc.subcore_barrier()

        base = s * ew
        s1 = ((dst0_v, exb0_v, gsem0), (dst1_v, exb1_v, gsem1))

        def phase1(i, slot, first):
            dv, exb, sem = s1[slot]
            if not first:
                pltpu.make_async_copy(exb, denom_sh.at[dv], sem).wait()
            off = base + i * block
            pltpu.sync_copy(e_hbm.at[pl.ds(off, block)], e_v)
            pltpu.sync_copy(dst_hbm.at[pl.ds(off, block)], dv)

            def grp(g, _):
                sl = pl.ds(g * L, L)
                exb[sl] = jnp.exp(e_v[sl])
                return 0

            lax.fori_loop(0, block // L, grp, 0)
            pltpu.async_copy(exb, denom_sh.at[dv], sem, add=True)

        phase1(0, 0, True)
        phase1(1, 1, True)

        def pair1(t, _):
            phase1(2 * t, 0, False)
            phase1(2 * t + 1, 1, False)
            return 0

        assert nchunks % 2 == 0
        lax.fori_loop(1, nchunks // 2, pair1, 0)
        pltpu.make_async_copy(exb0_v, denom_sh.at[dst0_v], gsem0).wait()
        pltpu.make_async_copy(exb1_v, denom_sh.at[dst1_v], gsem1).wait()
        plsc.subcore_barrier()
        pltpu.sync_copy(denom_sh, denom_v)

        if feature_split:
            base2 = s * ew2
        else:
            base2 = (c * NSUB + s) * ew2

        slots = ((src0_v, dst0_v, rows0_v, exb0_v, gsem0),
                 (src1_v, dst1_v, rows1_v, exb1_v, gsem1))

        def issue2(i, slot):
            sv, dv, rv, ev2, gsem = slots[slot]

            def do():
                off = base2 + i * block
                pltpu.sync_copy(src_hbm.at[pl.ds(off, block)], sv)
                pltpu.sync_copy(dst_hbm.at[pl.ds(off, block)], dv)
                pltpu.sync_copy(e_hbm.at[pl.ds(off, block)], ev2)
                if feature_split:
                    @pl.when(c == 0)
                    def _():
                        pltpu.async_copy(xla_hbm.at[sv], rv, gsem)

                    @pl.when(c == 1)
                    def _():
                        pltpu.async_copy(xlb_hbm.at[sv], rv, gsem)
                else:
                    pltpu.async_copy(xla_hbm.at[sv], rv, gsem)

            if isinstance(i, int):
                if i < nchunks2:
                    do()
            else:
                pl.when(i < nchunks2)(do)

        def phase2(i, slot, first):
            sv, dv, rv, ev2, gsem = slots[slot]
            pltpu.make_async_copy(xla_hbm.at[sv], rv, gsem).wait()

            def grp(g, _):
                sl = pl.ds(g * L, L)
                den = plsc.load_gather(denom_v, [dv[sl]])
                alpha_v[sl] = jnp.exp(ev2[sl]) / (den + 1e-16)
                return 0

            lax.fori_loop(0, block // L, grp, 0)
            if not first:
                pltpu.make_async_copy(srows_v, out_sh.at[sdst_v], ssem).wait()

            def edge_grp(gg, _):
                for jj in range(L):
                    j = gg * L + jj
                    ab = plsc.load_gather(alpha_v, [jnp.full((L,), j, jnp.int32)])
                    for cc in range(nfc):
                        sl = pl.ds(cc * L, L)
                        srows_v[j, sl] = rv[j, sl] * ab
                return 0

            lax.fori_loop(0, block // L, edge_grp, 0)

            def icp(g, _):
                sl = pl.ds(g * L, L)
                sdst_v[sl] = dv[sl]
                return 0

            lax.fori_loop(0, block // L, icp, 0)
            pltpu.async_copy(srows_v, out_sh.at[sdst_v], ssem, add=True)
            issue2(i + 2, slot)

        issue2(0, 0)
        issue2(1, 1)
        phase2(0, 0, True)
        phase2(1, 1, False)

        def pair2(t, _):
            phase2(2 * t, 0, False)
            phase2(2 * t + 1, 1, False)
            return 0

        if nchunks2 % 2 == 0:
            lax.fori_loop(1, nchunks2 // 2, pair2, 0)
        else:
            lax.fori_loop(1, (nchunks2 - 1) // 2, pair2, 0)
            phase2(nchunks2 - 1, 0, False)
        pltpu.make_async_copy(srows_v, out_sh.at[sdst_v], ssem).wait()
        plsc.subcore_barrier()

        for kk in range(rpt // cr):
            row0 = s * rpt + kk * cr

            @pl.when(c == 0)
            def _():
                pltpu.sync_copy(out_sh.at[pl.ds(row0, cr)],
                                outa_hbm.at[pl.ds(row0, cr)])

            @pl.when(c == 1)
            def _():
                pltpu.sync_copy(out_sh.at[pl.ds(row0, cr)],
                                outb_hbm.at[pl.ds(row0, cr)])

    return k(xla, xlb, e, src, dst)


def _sc_gather_rows_sum2(table_a, table_b, idx, block=128):
    n, f = table_a.shape
    m = idx.shape[0]
    per_w = m // NW
    nchunks = per_w // block
    nfc = f // L
    assert nchunks % 2 == 0

    @functools.partial(
        pl.kernel,
        out_type=jax.ShapeDtypeStruct((m, f), jnp.float32),
        mesh=_mesh(),
        compiler_params=pltpu.CompilerParams(needs_layout_passes=False),
        scratch_types=[
            pltpu.VMEM((block,), jnp.int32),
            pltpu.VMEM((block,), jnp.int32),
            pltpu.VMEM((block, f), jnp.float32),
            pltpu.VMEM((block, f), jnp.float32),
            pltpu.VMEM((block, f), jnp.float32),
            pltpu.VMEM((block, f), jnp.float32),
            pltpu.VMEM((block, f), jnp.float32),
            pltpu.VMEM((block, f), jnp.float32),
            pltpu.SemaphoreType.DMA,
            pltpu.SemaphoreType.DMA,
            pltpu.SemaphoreType.DMA,
            pltpu.SemaphoreType.DMA,
        ],
    )
    def k(taba_hbm, tabb_hbm, idx_hbm, out_hbm,
          idx0_v, idx1_v, ra0_v, ra1_v, rb0_v, rb1_v, sr0_v, sr1_v,
          gsem0, gsem1, osem0, osem1):
        wid = lax.axis_index("s") * NCORES + lax.axis_index("c")
        base = wid * per_w
        slots = ((idx0_v, ra0_v, rb0_v, sr0_v, gsem0, osem0),
                 (idx1_v, ra1_v, rb1_v, sr1_v, gsem1, osem1))

        def issue(i, slot):
            iv, ra, rb, _, gsem, _ = slots[slot]

            def do():
                off = base + i * block
                pltpu.sync_copy(idx_hbm.at[pl.ds(off, block)], iv)
                pltpu.async_copy(taba_hbm.at[iv], ra, gsem)
                pltpu.async_copy(tabb_hbm.at[iv], rb, gsem)

            if isinstance(i, int):
                if i < nchunks:
                    do()
            else:
                pl.when(i < nchunks)(do)

        def phase(i, slot, first):
            iv, ra, rb, sr, gsem, osem = slots[slot]
            off = base + i * block
            pltpu.make_async_copy(taba_hbm.at[iv], ra, gsem).wait()
            pltpu.make_async_copy(tabb_hbm.at[iv], rb, gsem).wait()
            if not first:
                pltpu.make_async_copy(sr, out_hbm.at[pl.ds(off, block)], osem).wait()

            def row_grp(gg, _):
                for jj in range(8):
                    j = gg * 8 + jj
                    for cc in range(nfc):
                        sl = pl.ds(cc * L, L)
                        sr[j, sl] = ra[j, sl] + rb[j, sl]
                return 0

            lax.fori_loop(0, block // 8, row_grp, 0)
            pltpu.async_copy(sr, out_hbm.at[pl.ds(off, block)], osem)
            issue(i + 2, slot)

        issue(0, 0)
        issue(1, 1)
        phase(0, 0, True)
        phase(1, 1, True)

        def pair(t, _):
            phase(2 * t, 0, False)
            phase(2 * t + 1, 1, False)
            return 0

        lax.fori_loop(1, nchunks // 2, pair, 0)
        pltpu.make_async_copy(sr0_v, out_hbm.at[pl.ds(base, block)], osem0).wait()
        pltpu.make_async_copy(sr1_v, out_hbm.at[pl.ds(base, block)], osem1).wait()

    return k(table_a, table_b, idx)


def kernel(x, edge_index, pos_edge_index, neg_edge_index,
           Wl1, Wr1, att1, bc1, Wl2, Wr2, att2, bc2,
           W1, b1, W2, b2, W3, b3, W4, b4):
    n = x.shape[0]
    n_pad = ((n + 2047) // 2048) * 2048
    src = edge_index[0].astype(jnp.int32)
    dst = edge_index[1].astype(jnp.int32)

    out1 = Wl1.shape[1]
    lr1 = _tc_matmul(x, jnp.concatenate([Wl1, Wr1], axis=1))
    xl1 = lr1[:, :out1]
    xr1 = lr1[:, out1:]
    e1 = _sc_edge_scores(xl1, xr1, att1, src, dst)
    h1dim = out1 // 2
    outa1, outb1 = _sc_softmax_agg(xl1[:, :h1dim], xl1[:, h1dim:], e1, src, dst,
                                   n_pad, feature_split=True)
    agg1 = jnp.concatenate([outa1[:n], outb1[:n]], axis=1)

    out2 = Wl2.shape[1]
    lr2 = _tc_matmul(agg1, jnp.concatenate([Wl2, Wr2], axis=1),
                     pre_bias=bc1, pre_relu=True)
    xl2 = lr2[:, :out2]
    xr2 = lr2[:, out2:]
    e2 = _sc_edge_scores(xl2, xr2, att2, src, dst)
    outa2, outb2 = _sc_softmax_agg(xl2, xl2, e2, src, dst,
                                   n_pad, feature_split=False)

    idx_all = jnp.concatenate([pos_edge_index[0], neg_edge_index[0],
                               pos_edge_index[1], neg_edge_index[1]]).astype(jnp.int32)
    rows = _sc_gather_rows_sum2(outa2, outb2, idx_all)
    m = idx_all.shape[0] // 2
    h0 = rows[:m]
    h1 = rows[m:]

    kdim = W1.shape[0] // 2
    w1a = W1[:kdim]
    w1b = W1[kdim:]
    w4p = jnp.pad(W4, ((0, 0), (0, 127)))
    b4p = jnp.pad(b4, (0, 127))
    mlp = _tc_decode_mlp(h0, h1, bc2, w1a, w1b, b1, W2, b2, W3, b3, w4p, b4p)
    return mlp[:, 0]
